# Initial kernel scaffold; baseline (speedup 1.0000x reference)
import jax
import jax.numpy as jnp
from jax.experimental import pallas as pl


def kernel(n_id, local_node_idx, edge_index, edge_type, node_type, emb, params):
    raise NotImplementedError("write your pallas kernel here")


# speedup vs baseline: 11.9053x; 11.9053x over previous
import functools
import jax
import jax.numpy as jnp
from jax import lax
from jax.experimental import pallas as pl
from jax.experimental.pallas import tpu as pltpu
from jax.experimental.pallas import tpu_sc as plsc

N = 10000
E = 320000
R = 5
H = 2
NEG = 0.2

NC, NS, L = 2, 16, 16
NW = NC * NS
NP = 10240
KS = 51200
TRK = KS // NS
TRD = NP // NS

f32 = jnp.float32
i32 = jnp.int32


def _mesh():
    return plsc.VectorSubcoreMesh(
        core_axis_name="c", subcore_axis_name="s",
        num_cores=NC, num_subcores=NS)


def _lane_take(v, idx):
    return lax.gather(
        v, idx[:, None],
        lax.GatherDimensionNumbers(offset_dims=(), collapsed_slice_dims=(0,),
                                   start_index_map=(0,)),
        (1,), mode=lax.GatherScatterMode.PROMISE_IN_BOUNDS)


@functools.partial(
    pl.kernel, mesh=_mesh(),
    compiler_params=pltpu.CompilerParams(use_tc_tiling_on_sc=False),
    out_type=jax.ShapeDtypeStruct((NP, 128), f32),
    scratch_types=[
        pltpu.VMEM((128,), i32),
        pltpu.VMEM((128, 128), f32),
        pltpu.SemaphoreType.DMA,
    ])
def _gather_rows(emb_hbm, idx_hbm, out_hbm, idx_v, rows_v, sem):
    wid = lax.axis_index("s") * NC + lax.axis_index("c")
    nb = (NP // 128 - wid + NW - 1) // NW

    def blk(b, _):
        base = (b * NW + wid) * 128
        pltpu.sync_copy(idx_hbm.at[pl.ds(base, 128)], idx_v)
        pltpu.async_copy(emb_hbm.at[idx_v], rows_v, sem).wait()
        pltpu.sync_copy(rows_v, out_hbm.at[pl.ds(base, 128)])
        return _
    lax.fori_loop(0, nb, blk, None)


BA = 512
NBLK = E // BA


def _attn_body(src_h, dst_h, et_h, ai_h, aj_h,
               exr_h, key_h, denp_h,
               den_sh, src_v, dst_v, et_v, key_v,
               ai_r, aj_r, exr_r, s1, s2):
    c = lax.axis_index("c")
    t = lax.axis_index("s")
    lanes = lax.iota(i32, L)

    def zrow(i, _):
        exr_r.at[i][...] = jnp.zeros((L,), f32)
        return _
    lax.fori_loop(0, BA, zrow, None)
    pltpu.sync_copy(exr_r, den_sh.at[pl.ds(t * TRD, BA)])
    pltpu.sync_copy(exr_r.at[pl.ds(0, TRD - BA)],
                    den_sh.at[pl.ds(t * TRD + BA, TRD - BA)])
    plsc.subcore_barrier()

    wid = c * NS + t

    def block(b, _):
        base = (b * NW + wid) * BA
        pltpu.sync_copy(src_h.at[pl.ds(base, BA)], src_v)
        pltpu.sync_copy(dst_h.at[pl.ds(base, BA)], dst_v)
        pltpu.sync_copy(et_h.at[pl.ds(base, BA)], et_v)

        def keyloop(j, _):
            sl = pl.ds(j * L, L)
            key_v[sl] = et_v[sl] * N + dst_v[sl]
            return _
        lax.fori_loop(0, BA // L, keyloop, None)
        cp1 = pltpu.async_copy(ai_h.at[dst_v], ai_r, s1)
        cp2 = pltpu.async_copy(aj_h.at[src_v], aj_r, s2)
        cp1.wait()
        cp2.wait()

        def exloop(j, _):
            etv = et_v[pl.ds(j * L, L)]
            for l in range(L):
                e = j * L + l
                col = _lane_take(etv, jnp.full((L,), l, i32)) * H
                v = ai_r.at[e][...] + aj_r.at[e][...]
                exv = jnp.exp(jnp.maximum(v, NEG * v))
                keep = (lanes == col) | (lanes == col + 1)
                exr_r.at[e][...] = jnp.where(keep, exv, 0.0)
            return _
        lax.fori_loop(0, BA // L, exloop, None)
        pltpu.sync_copy(exr_r, den_sh.at[dst_v], add=True)
        pltpu.sync_copy(exr_r, exr_h.at[pl.ds(base, BA)])
        pltpu.sync_copy(key_v, key_h.at[pl.ds(base, BA)])
        return _
    lax.fori_loop(0, (NBLK - wid + NW - 1) // NW, block, None)

    plsc.subcore_barrier()
    pltpu.sync_copy(den_sh.at[pl.ds(t * TRD, TRD)],
                    denp_h.at[c].at[pl.ds(t * TRD, TRD)])


def _make_attn():
    return pl.kernel(
        _attn_body,
        out_type=(jax.ShapeDtypeStruct((E, 16), f32),
                  jax.ShapeDtypeStruct((E,), i32),
                  jax.ShapeDtypeStruct((NC, NP, 16), f32)),
        mesh=_mesh(),
        compiler_params=pltpu.CompilerParams(use_tc_tiling_on_sc=False),
        scratch_types=[
            pltpu.VMEM_SHARED((NP, 16), f32),
            pltpu.VMEM((BA,), i32), pltpu.VMEM((BA,), i32),
            pltpu.VMEM((BA,), i32), pltpu.VMEM((BA,), i32),
            pltpu.VMEM((BA, 16), f32), pltpu.VMEM((BA, 16), f32),
            pltpu.VMEM((BA, 16), f32),
            pltpu.SemaphoreType.DMA, pltpu.SemaphoreType.DMA,
        ])


BB = 512


def _msg_body(key_h, src_h, exr_h, hj0_h, hj1_h, hj2_h, hj3_h,
              zp_h,
              z_sh, key_v, src_v, exm_r, hj_r, s1, s2):
    c = lax.axis_index("c")
    t = lax.axis_index("s")
    hj_tabs = (hj0_h, hj1_h, hj2_h, hj3_h)
    lanes = lax.iota(i32, L)

    for cc in range(4):
        h = cc // 2
        par = (lanes % 2 == h)

        @pl.when(c == h)
        def _chunk():
            def zrow(i, _):
                hj_r.at[i][pl.ds(0, L)] = jnp.zeros((L,), f32)
                hj_r.at[i][pl.ds(L, L)] = jnp.zeros((L,), f32)
                return _
            lax.fori_loop(0, BB, zrow, None)
            for k in range(6):
                pltpu.sync_copy(hj_r, z_sh.at[pl.ds(t * TRK + k * BB, BB)])
            pltpu.sync_copy(hj_r.at[pl.ds(0, 128)],
                            z_sh.at[pl.ds(t * TRK + 6 * BB, 128)])
            plsc.subcore_barrier()

            def block(b, _):
                base = (b * NS + t) * BB
                pltpu.sync_copy(key_h.at[pl.ds(base, BB)], key_v)
                pltpu.sync_copy(src_h.at[pl.ds(base, BB)], src_v)
                cp1 = pltpu.async_copy(exr_h.at[pl.ds(base, BB)], exm_r, s1)
                cp2 = pltpu.async_copy(hj_tabs[cc].at[src_v], hj_r, s2)
                cp1.wait()
                cp2.wait()

                def scale(e, _):
                    exm = exm_r.at[e][...]
                    sv = jnp.where(par, exm, 0.0)
                    for sh in (8, 4, 2, 1):
                        sv = sv + _lane_take(sv, lanes ^ sh)
                    r0 = hj_r.at[e][pl.ds(0, L)]
                    r1 = hj_r.at[e][pl.ds(L, L)]
                    hj_r.at[e][pl.ds(0, L)] = r0 * sv
                    hj_r.at[e][pl.ds(L, L)] = r1 * sv
                    return _
                lax.fori_loop(0, BB, scale, None)
                pltpu.sync_copy(hj_r, z_sh.at[key_v], add=True)
                return _
            lax.fori_loop(0, (NBLK - t + NS - 1) // NS, block, None)

            plsc.subcore_barrier()
            pltpu.sync_copy(z_sh.at[pl.ds(t * TRK, TRK)],
                            zp_h.at[cc].at[pl.ds(t * TRK, TRK)])
            plsc.subcore_barrier()


def _make_msg():
    return pl.kernel(
        _msg_body,
        out_type=jax.ShapeDtypeStruct((4, KS, 32), f32),
        mesh=_mesh(),
        compiler_params=pltpu.CompilerParams(use_tc_tiling_on_sc=False),
        scratch_types=[
            pltpu.VMEM_SHARED((KS, 32), f32),
            pltpu.VMEM((BB,), i32), pltpu.VMEM((BB,), i32),
            pltpu.VMEM((BB, 16), f32), pltpu.VMEM((BB, 32), f32),
            pltpu.SemaphoreType.DMA, pltpu.SemaphoreType.DMA,
        ])


def _pre_tc(x, wcat):
    M, K = x.shape
    bn = 1024
    grid = (M // bn,)

    def body(x_ref, w_ref, hj0, hj1, hj2, hj3, sn, ai, aj, xs):
        y = jnp.dot(x_ref[...], w_ref[...], preferred_element_type=f32)
        hj0[...] = y[:, 0:32]
        hj1[...] = y[:, 32:64]
        hj2[...] = y[:, 64:96]
        hj3[...] = y[:, 96:128]
        sn[...] = y[:, 128:256]
        ai[...] = y[:, 256:272]
        aj[...] = y[:, 272:288]
        xs[...] = y[:, 288:352]

    outs = [jax.ShapeDtypeStruct((M, 32), f32)] * 4 + [
        jax.ShapeDtypeStruct((M, 128), f32),
        jax.ShapeDtypeStruct((M, 16), f32),
        jax.ShapeDtypeStruct((M, 16), f32),
        jax.ShapeDtypeStruct((M, 64), f32),
    ]
    ospec = [pl.BlockSpec((bn, s.shape[1]), lambda i: (i, 0)) for s in outs]
    return pl.pallas_call(
        body,
        grid=grid,
        in_specs=[pl.BlockSpec((bn, K), lambda i: (i, 0)),
                  pl.BlockSpec((K, 352), lambda i: (0, 0))],
        out_specs=ospec,
        out_shape=outs,
    )(x, wcat)


def _post_tc(zp, denp, sn, xs, wqkv, wrel, last):
    bn = 400
    grid = (N // bn,)

    def body(z0, z1, z2, z3, z4, dn_r, sn_r, xs_r, wq_r, wr_r, out):
        zrefs = (z0, z1, z2, z3, z4)
        den = dn_r[0] + dn_r[1]
        den = jnp.where(den > 0, den, 1.0)
        sn_b = sn_r[...]
        xs_b = xs_r[...]
        qs, ks, vs = [], [], []
        for r in range(R):
            zr = zrefs[r][...]
            d0 = den[:, 2 * r:2 * r + 1]
            d1 = den[:, 2 * r + 1:2 * r + 2]
            zcat = jnp.concatenate(
                [zr[0] / d0, zr[1] / d0, zr[2] / d1, zr[3] / d1], axis=1)
            zfull = zcat + sn_b
            qkv = jnp.dot(zfull, wq_r[r], preferred_element_type=f32)
            qs.append(qkv[:, 0:64])
            ks.append(qkv[:, 64:128])
            vs.append(qkv[:, 128:192])
        acc = jnp.zeros((bn, 64), f32)
        for r in range(R):
            es = [jnp.exp(jnp.sum(qs[r] * ks[s2], axis=1, keepdims=True))
                  for s2 in range(R)]
            tot = es[0] + es[1] + es[2] + es[3] + es[4]
            delta = sum((es[s2] / tot) * vs[s2] for s2 in range(R))
            m = (jnp.sum(delta, axis=1, keepdims=True) != 0).astype(f32)
            acc = acc + (delta + xs_b * m) * wr_r[r:r + 1, 0:1]
        if last:
            mx = jnp.max(acc, axis=1, keepdims=True)
            lse = mx + jnp.log(jnp.sum(jnp.exp(acc - mx), axis=1,
                                       keepdims=True))
            out[...] = acc - lse
        else:
            out[...] = jnp.maximum(acc, 0.0)

    def zspec(r):
        return pl.BlockSpec((4, bn, 32),
                            lambda i, r=r: (0, r * (N // bn) + i, 0))

    return pl.pallas_call(
        body,
        grid=grid,
        in_specs=[zspec(0), zspec(1), zspec(2), zspec(3), zspec(4),
                  pl.BlockSpec((NC, bn, 16), lambda i: (0, i, 0)),
                  pl.BlockSpec((bn, 128), lambda i: (i, 0)),
                  pl.BlockSpec((bn, 64), lambda i: (i, 0)),
                  pl.BlockSpec((R, 128, 192), lambda i: (0, 0, 0)),
                  pl.BlockSpec((8, 128), lambda i: (0, 0))],
        out_specs=pl.BlockSpec((bn, 64), lambda i: (i, 0)),
        out_shape=jax.ShapeDtypeStruct((N, 64), f32),
    )(zp, zp, zp, zp, zp, denp, sn, xs, wqkv, wrel)


def _fold_weights(p, out_c):
    att = p['node_att']
    att_i = att[:, :, :out_c]
    att_j = att[:, :, out_c:]
    eye = jnp.eye(H, dtype=f32)
    Mi = jnp.einsum('rhc,hk->hcrk', att_i, eye).reshape(H * out_c, R * H)
    Mj = jnp.einsum('rhc,hk->hcrk', att_j, eye).reshape(H * out_c, R * H)
    pad = jnp.zeros((p['lin_i'].shape[0], 16 - R * H), f32)
    w_ai = jnp.concatenate([p['lin_i'] @ Mi, pad], axis=1)
    w_aj = jnp.concatenate([p['lin_j'] @ Mj, pad], axis=1)
    wcat = jnp.concatenate(
        [p['lin_j'], p['W_self_node'], w_ai, w_aj, p['W_self']], axis=1)
    wqkv = jnp.concatenate([p['W_q'], p['W_k'], p['W_v']], axis=2)
    wrel = jnp.zeros((8, 128), f32).at[:R, 0].set(p['W_relation'][:, 0])
    return wcat, wqkv, wrel


_attn_call = _make_attn()
_msg_call = _make_msg()


def kernel(n_id, local_node_idx, edge_index, edge_type, node_type, emb,
           params):
    src = edge_index[0]
    dst = edge_index[1]
    wcat1, wqkv1, wrel1 = _fold_weights(params[0], 64)
    wcat2, wqkv2, wrel2 = _fold_weights(params[1], 64)

    idxp = jnp.concatenate([local_node_idx, jnp.zeros((NP - N,), i32)])
    x = _gather_rows(emb, idxp)

    for li, (wcat, wqkv, wrel) in enumerate(
            [(wcat1, wqkv1, wrel1), (wcat2, wqkv2, wrel2)]):
        hj0, hj1, hj2, hj3, sn, ai, aj, xs = _pre_tc(x, wcat)
        exr, key, denp = _attn_call(src, dst, edge_type, ai, aj)
        zp = _msg_call(key, src, exr, hj0, hj1, hj2, hj3)
        x = _post_tc(zp, denp, sn, xs, wqkv, wrel, last=(li == 1))
        if li == 0:
            x = jnp.concatenate([x, jnp.zeros((NP - N, 64), f32)], axis=0)
    return x


# speedup vs baseline: 16.1500x; 1.3565x over previous
import functools
import jax
import jax.numpy as jnp
from jax import lax
from jax.experimental import pallas as pl
from jax.experimental.pallas import tpu as pltpu
from jax.experimental.pallas import tpu_sc as plsc

N = 10000
E = 320000
R = 5
H = 2
NEG = 0.2

NC, NS, L = 2, 16, 16
NW = NC * NS
NP = 10240
KS = 50048
TRK = KS // NS
TRD = NP // NS

f32 = jnp.float32
i32 = jnp.int32


def _mesh():
    return plsc.VectorSubcoreMesh(
        core_axis_name="c", subcore_axis_name="s",
        num_cores=NC, num_subcores=NS)


def _lane_take(v, idx):
    return lax.gather(
        v, idx[:, None],
        lax.GatherDimensionNumbers(offset_dims=(), collapsed_slice_dims=(0,),
                                   start_index_map=(0,)),
        (1,), mode=lax.GatherScatterMode.PROMISE_IN_BOUNDS)


@functools.partial(
    pl.kernel, mesh=_mesh(),
    compiler_params=pltpu.CompilerParams(use_tc_tiling_on_sc=False),
    out_type=jax.ShapeDtypeStruct((NP, 128), f32),
    scratch_types=[
        pltpu.VMEM((128,), i32),
        pltpu.VMEM((128, 128), f32),
        pltpu.SemaphoreType.DMA,
    ])
def _gather_rows(emb_hbm, idx_hbm, out_hbm, idx_v, rows_v, sem):
    wid = lax.axis_index("s") * NC + lax.axis_index("c")
    nb = (NP // 128 - wid + NW - 1) // NW

    def blk(b, _):
        base = (b * NW + wid) * 128
        pltpu.sync_copy(idx_hbm.at[pl.ds(base, 128)], idx_v)
        pltpu.async_copy(emb_hbm.at[idx_v], rows_v, sem).wait()
        pltpu.sync_copy(rows_v, out_hbm.at[pl.ds(base, 128)])
        return _
    lax.fori_loop(0, nb, blk, None)


BA = 512
NBLK = E // BA


def _attn_body(src_h, dst_h, et_h, ai_h, aj_h,
               exp_h, key_h, denp_h,
               den_sh, src_v, dst_v, et_v, key_v,
               ai_r, aj_r, exr_r, exp0_v, exp1_v, s1, s2):
    c = lax.axis_index("c")
    t = lax.axis_index("s")
    lanes = lax.iota(i32, L)

    def zrow(i, _):
        exr_r.at[i][...] = jnp.zeros((L,), f32)
        return _
    lax.fori_loop(0, BA, zrow, None)
    pltpu.sync_copy(exr_r, den_sh.at[pl.ds(t * TRD, BA)])
    pltpu.sync_copy(exr_r.at[pl.ds(0, TRD - BA)],
                    den_sh.at[pl.ds(t * TRD + BA, TRD - BA)])
    plsc.subcore_barrier()

    wid = c * NS + t

    def block(b, _):
        base = (b * NW + wid) * BA
        pltpu.sync_copy(src_h.at[pl.ds(base, BA)], src_v)
        pltpu.sync_copy(dst_h.at[pl.ds(base, BA)], dst_v)
        pltpu.sync_copy(et_h.at[pl.ds(base, BA)], et_v)

        def keyloop(j, _):
            sl = pl.ds(j * L, L)
            key_v[sl] = et_v[sl] * N + dst_v[sl]
            return _
        lax.fori_loop(0, BA // L, keyloop, None)
        cp1 = pltpu.async_copy(ai_h.at[dst_v], ai_r, s1)
        cp2 = pltpu.async_copy(aj_h.at[src_v], aj_r, s2)
        cp1.wait()
        cp2.wait()

        def exloop(j, _):
            etv = et_v[pl.ds(j * L, L)]
            acc0 = jnp.zeros((L,), f32)
            acc1 = jnp.zeros((L,), f32)
            for l in range(L):
                e = j * L + l
                col = _lane_take(etv, jnp.full((L,), l, i32)) * H
                v = ai_r.at[e][...] + aj_r.at[e][...]
                exv = jnp.exp(jnp.maximum(v, NEG * v))
                keep = (lanes == col) | (lanes == col + 1)
                exm = jnp.where(keep, exv, 0.0)
                exr_r.at[e][...] = exm
                sv = exm
                for sh in (8, 4, 2):
                    sv = sv + _lane_take(sv, lanes ^ sh)
                sx = _lane_take(sv, lanes ^ 1)
                acc0 = jnp.where(lanes == l, sv if l % 2 == 0 else sx, acc0)
                acc1 = jnp.where(lanes == l, sx if l % 2 == 0 else sv, acc1)
            exp0_v[pl.ds(j * L, L)] = acc0
            exp1_v[pl.ds(j * L, L)] = acc1
            return _
        lax.fori_loop(0, BA // L, exloop, None)
        pltpu.sync_copy(exr_r, den_sh.at[dst_v], add=True)
        pltpu.sync_copy(exp0_v, exp_h.at[0].at[pl.ds(base, BA)])
        pltpu.sync_copy(exp1_v, exp_h.at[1].at[pl.ds(base, BA)])
        pltpu.sync_copy(key_v, key_h.at[pl.ds(base, BA)])
        return _
    lax.fori_loop(0, (NBLK - wid + NW - 1) // NW, block, None)

    plsc.subcore_barrier()
    pltpu.sync_copy(den_sh.at[pl.ds(t * TRD, TRD)],
                    denp_h.at[c].at[pl.ds(t * TRD, TRD)])


def _make_attn():
    return pl.kernel(
        _attn_body,
        out_type=(jax.ShapeDtypeStruct((2, E), f32),
                  jax.ShapeDtypeStruct((E,), i32),
                  jax.ShapeDtypeStruct((NC, NP, 16), f32)),
        mesh=_mesh(),
        compiler_params=pltpu.CompilerParams(use_tc_tiling_on_sc=False),
        scratch_types=[
            pltpu.VMEM_SHARED((NP, 16), f32),
            pltpu.VMEM((BA,), i32), pltpu.VMEM((BA,), i32),
            pltpu.VMEM((BA,), i32), pltpu.VMEM((BA,), i32),
            pltpu.VMEM((BA, 16), f32), pltpu.VMEM((BA, 16), f32),
            pltpu.VMEM((BA, 16), f32),
            pltpu.VMEM((BA,), f32), pltpu.VMEM((BA,), f32),
            pltpu.SemaphoreType.DMA, pltpu.SemaphoreType.DMA,
        ])


BB = 256
NBLKB = E // BB


def _msg_body(key_h, src_h, exp_h, hj0_h, hj1_h, hj2_h, hj3_h,
              zp_h,
              z_sh, hj_sh, key_v, src_v, exm_v, hj_r, s1, s2):
    c = lax.axis_index("c")
    t = lax.axis_index("s")
    hj_tabs = (hj0_h, hj1_h, hj2_h, hj3_h)
    lanes = lax.iota(i32, L)

    for cc in range(4):
        h = cc // 2

        @pl.when(c == h)
        def _chunk():
            def zrow(i, _):
                hj_r.at[i][pl.ds(0, L)] = jnp.zeros((L,), f32)
                hj_r.at[i][pl.ds(L, L)] = jnp.zeros((L,), f32)
                return _
            lax.fori_loop(0, BB, zrow, None)
            for k in range(12):
                pltpu.sync_copy(hj_r, z_sh.at[pl.ds(t * TRK + k * BB, BB)])
            pltpu.sync_copy(hj_r.at[pl.ds(0, 56)],
                            z_sh.at[pl.ds(t * TRK + 12 * BB, 56)])
            pltpu.sync_copy(hj_tabs[cc].at[pl.ds(t * TRD, TRD)],
                            hj_sh.at[pl.ds(t * TRD, TRD)])
            plsc.subcore_barrier()

            def block(b, _):
                base = (b * NS + t) * BB
                pltpu.sync_copy(key_h.at[pl.ds(base, BB)], key_v)
                pltpu.sync_copy(src_h.at[pl.ds(base, BB)], src_v)
                cp1 = pltpu.async_copy(exp_h.at[h].at[pl.ds(base, BB)],
                                       exm_v, s1)
                cp2 = pltpu.async_copy(hj_sh.at[src_v], hj_r, s2)
                cp1.wait()
                cp2.wait()

                def scale(j, _):
                    row = exm_v[pl.ds(j * L, L)]
                    for l in range(L):
                        e = j * L + l
                        sv = _lane_take(row, jnp.full((L,), l, i32))
                        r0 = hj_r.at[e][pl.ds(0, L)]
                        r1 = hj_r.at[e][pl.ds(L, L)]
                        hj_r.at[e][pl.ds(0, L)] = r0 * sv
                        hj_r.at[e][pl.ds(L, L)] = r1 * sv
                    return _
                lax.fori_loop(0, BB // L, scale, None)
                pltpu.sync_copy(hj_r, z_sh.at[key_v], add=True)
                return _
            lax.fori_loop(0, (NBLKB - t + NS - 1) // NS, block, None)

            plsc.subcore_barrier()
            pltpu.sync_copy(z_sh.at[pl.ds(t * TRK, TRK)],
                            zp_h.at[cc].at[pl.ds(t * TRK, TRK)])
            plsc.subcore_barrier()


def _make_msg():
    return pl.kernel(
        _msg_body,
        out_type=jax.ShapeDtypeStruct((4, KS, 32), f32),
        mesh=_mesh(),
        compiler_params=pltpu.CompilerParams(use_tc_tiling_on_sc=False),
        scratch_types=[
            pltpu.VMEM_SHARED((KS, 32), f32),
            pltpu.VMEM_SHARED((NP, 32), f32),
            pltpu.VMEM((BB,), i32), pltpu.VMEM((BB,), i32),
            pltpu.VMEM((BB,), f32), pltpu.VMEM((BB, 32), f32),
            pltpu.SemaphoreType.DMA, pltpu.SemaphoreType.DMA,
        ])


def _pre_tc(x, wcat):
    M, K = x.shape
    bn = 1024
    grid = (M // bn,)

    def body(x_ref, w_ref, hj0, hj1, hj2, hj3, sn, ai, aj, xs):
        y = jnp.dot(x_ref[...], w_ref[...], preferred_element_type=f32)
        hj0[...] = y[:, 0:32]
        hj1[...] = y[:, 32:64]
        hj2[...] = y[:, 64:96]
        hj3[...] = y[:, 96:128]
        sn[...] = y[:, 128:256]
        ai[...] = y[:, 256:272]
        aj[...] = y[:, 272:288]
        xs[...] = y[:, 288:352]

    outs = [jax.ShapeDtypeStruct((M, 32), f32)] * 4 + [
        jax.ShapeDtypeStruct((M, 128), f32),
        jax.ShapeDtypeStruct((M, 16), f32),
        jax.ShapeDtypeStruct((M, 16), f32),
        jax.ShapeDtypeStruct((M, 64), f32),
    ]
    ospec = [pl.BlockSpec((bn, s.shape[1]), lambda i: (i, 0)) for s in outs]
    return pl.pallas_call(
        body,
        grid=grid,
        in_specs=[pl.BlockSpec((bn, K), lambda i: (i, 0)),
                  pl.BlockSpec((K, 352), lambda i: (0, 0))],
        out_specs=ospec,
        out_shape=outs,
    )(x, wcat)


def _post_tc(zp, denp, sn, xs, wqkv, wrel, last):
    bn = 400
    grid = (N // bn,)

    def body(z0, z1, z2, z3, z4, dn_r, sn_r, xs_r, wq_r, wr_r, out):
        zrefs = (z0, z1, z2, z3, z4)
        den = dn_r[0] + dn_r[1]
        den = jnp.where(den > 0, den, 1.0)
        sn_b = sn_r[...]
        xs_b = xs_r[...]
        qs, ks, vs = [], [], []
        for r in range(R):
            zr = zrefs[r][...]
            d0 = den[:, 2 * r:2 * r + 1]
            d1 = den[:, 2 * r + 1:2 * r + 2]
            zcat = jnp.concatenate(
                [zr[0] / d0, zr[1] / d0, zr[2] / d1, zr[3] / d1], axis=1)
            zfull = zcat + sn_b
            qkv = jnp.dot(zfull, wq_r[r], preferred_element_type=f32)
            qs.append(qkv[:, 0:64])
            ks.append(qkv[:, 64:128])
            vs.append(qkv[:, 128:192])
        acc = jnp.zeros((bn, 64), f32)
        for r in range(R):
            es = [jnp.exp(jnp.sum(qs[r] * ks[s2], axis=1, keepdims=True))
                  for s2 in range(R)]
            tot = es[0] + es[1] + es[2] + es[3] + es[4]
            delta = sum((es[s2] / tot) * vs[s2] for s2 in range(R))
            m = (jnp.sum(delta, axis=1, keepdims=True) != 0).astype(f32)
            acc = acc + (delta + xs_b * m) * wr_r[r:r + 1, 0:1]
        if last:
            mx = jnp.max(acc, axis=1, keepdims=True)
            lse = mx + jnp.log(jnp.sum(jnp.exp(acc - mx), axis=1,
                                       keepdims=True))
            out[...] = acc - lse
        else:
            out[...] = jnp.maximum(acc, 0.0)

    def zspec(r):
        return pl.BlockSpec((4, bn, 32),
                            lambda i, r=r: (0, r * (N // bn) + i, 0))

    return pl.pallas_call(
        body,
        grid=grid,
        in_specs=[zspec(0), zspec(1), zspec(2), zspec(3), zspec(4),
                  pl.BlockSpec((NC, bn, 16), lambda i: (0, i, 0)),
                  pl.BlockSpec((bn, 128), lambda i: (i, 0)),
                  pl.BlockSpec((bn, 64), lambda i: (i, 0)),
                  pl.BlockSpec((R, 128, 192), lambda i: (0, 0, 0)),
                  pl.BlockSpec((8, 128), lambda i: (0, 0))],
        out_specs=pl.BlockSpec((bn, 64), lambda i: (i, 0)),
        out_shape=jax.ShapeDtypeStruct((N, 64), f32),
    )(zp, zp, zp, zp, zp, denp, sn, xs, wqkv, wrel)


def _fold_weights(p, out_c):
    att = p['node_att']
    att_i = att[:, :, :out_c]
    att_j = att[:, :, out_c:]
    eye = jnp.eye(H, dtype=f32)
    Mi = jnp.einsum('rhc,hk->hcrk', att_i, eye).reshape(H * out_c, R * H)
    Mj = jnp.einsum('rhc,hk->hcrk', att_j, eye).reshape(H * out_c, R * H)
    pad = jnp.zeros((p['lin_i'].shape[0], 16 - R * H), f32)
    w_ai = jnp.concatenate([p['lin_i'] @ Mi, pad], axis=1)
    w_aj = jnp.concatenate([p['lin_j'] @ Mj, pad], axis=1)
    wcat = jnp.concatenate(
        [p['lin_j'], p['W_self_node'], w_ai, w_aj, p['W_self']], axis=1)
    wqkv = jnp.concatenate([p['W_q'], p['W_k'], p['W_v']], axis=2)
    wrel = jnp.zeros((8, 128), f32).at[:R, 0].set(p['W_relation'][:, 0])
    return wcat, wqkv, wrel


_attn_call = _make_attn()
_msg_call = _make_msg()


def kernel(n_id, local_node_idx, edge_index, edge_type, node_type, emb,
           params):
    src = edge_index[0]
    dst = edge_index[1]
    wcat1, wqkv1, wrel1 = _fold_weights(params[0], 64)
    wcat2, wqkv2, wrel2 = _fold_weights(params[1], 64)

    idxp = jnp.concatenate([local_node_idx, jnp.zeros((NP - N,), i32)])
    x = _gather_rows(emb, idxp)

    for li, (wcat, wqkv, wrel) in enumerate(
            [(wcat1, wqkv1, wrel1), (wcat2, wqkv2, wrel2)]):
        hj0, hj1, hj2, hj3, sn, ai, aj, xs = _pre_tc(x, wcat)
        exr, key, denp = _attn_call(src, dst, edge_type, ai, aj)
        zp = _msg_call(key, src, exr, hj0, hj1, hj2, hj3)
        zp = zp[:, :R * N, :]
        x = _post_tc(zp, denp, sn, xs, wqkv, wrel, last=(li == 1))
        if li == 0:
            x = jnp.concatenate([x, jnp.zeros((NP - N, 64), f32)], axis=0)
    return x


# speedup vs baseline: 16.7606x; 1.0378x over previous
import functools
import jax
import jax.numpy as jnp
from jax import lax
from jax.experimental import pallas as pl
from jax.experimental.pallas import tpu as pltpu
from jax.experimental.pallas import tpu_sc as plsc

N = 10000
E = 320000
R = 5
H = 2
NEG = 0.2

NC, NS, L = 2, 16, 16
NW = NC * NS
NP = 10240
KS = 50048
TRK = KS // NS
TRD = NP // NS

f32 = jnp.float32
i32 = jnp.int32


def _mesh():
    return plsc.VectorSubcoreMesh(
        core_axis_name="c", subcore_axis_name="s",
        num_cores=NC, num_subcores=NS)


def _lane_take(v, idx):
    return lax.gather(
        v, idx[:, None],
        lax.GatherDimensionNumbers(offset_dims=(), collapsed_slice_dims=(0,),
                                   start_index_map=(0,)),
        (1,), mode=lax.GatherScatterMode.PROMISE_IN_BOUNDS)


@functools.partial(
    pl.kernel, mesh=_mesh(),
    compiler_params=pltpu.CompilerParams(use_tc_tiling_on_sc=False),
    out_type=jax.ShapeDtypeStruct((NP, 128), f32),
    scratch_types=[
        pltpu.VMEM((128,), i32),
        pltpu.VMEM((128, 128), f32),
        pltpu.SemaphoreType.DMA,
    ])
def _gather_rows(emb_hbm, idx_hbm, out_hbm, idx_v, rows_v, sem):
    wid = lax.axis_index("s") * NC + lax.axis_index("c")
    nb = (NP // 128 - wid + NW - 1) // NW

    def blk(b, _):
        base = (b * NW + wid) * 128
        pltpu.sync_copy(idx_hbm.at[pl.ds(base, 128)], idx_v)
        pltpu.async_copy(emb_hbm.at[idx_v], rows_v, sem).wait()
        pltpu.sync_copy(rows_v, out_hbm.at[pl.ds(base, 128)])
        return _
    lax.fori_loop(0, nb, blk, None)


BA = 512
NBLK = E // BA


def _attn_body(src_h, dst_h, et_h, ai_h, aj_h,
               exp_h, key_h, denp_h,
               den_sh, src_v, dst_v, et_v, key_v,
               ai_r, aj_r, exr_r, exp0_v, exp1_v, s1, s2):
    c = lax.axis_index("c")
    t = lax.axis_index("s")
    lanes = lax.iota(i32, L)

    def zrow(i, _):
        exr_r.at[i][...] = jnp.zeros((L,), f32)
        return _
    lax.fori_loop(0, BA, zrow, None)
    pltpu.sync_copy(exr_r, den_sh.at[pl.ds(t * TRD, BA)])
    pltpu.sync_copy(exr_r.at[pl.ds(0, TRD - BA)],
                    den_sh.at[pl.ds(t * TRD + BA, TRD - BA)])
    plsc.subcore_barrier()

    wid = c * NS + t

    def block(b, _):
        base = (b * NW + wid) * BA
        pltpu.sync_copy(src_h.at[pl.ds(base, BA)], src_v)
        pltpu.sync_copy(dst_h.at[pl.ds(base, BA)], dst_v)
        pltpu.sync_copy(et_h.at[pl.ds(base, BA)], et_v)

        def keyloop(j, _):
            sl = pl.ds(j * L, L)
            key_v[sl] = et_v[sl] * N + dst_v[sl]
            return _
        lax.fori_loop(0, BA // L, keyloop, None)
        cp1 = pltpu.async_copy(ai_h.at[dst_v], ai_r, s1)
        cp2 = pltpu.async_copy(aj_h.at[src_v], aj_r, s2)
        cp1.wait()
        cp2.wait()

        landc = lanes & -2

        def exloop(j, _):
            etv = et_v[pl.ds(j * L, L)]
            acc0 = jnp.zeros((L,), f32)
            acc1 = jnp.zeros((L,), f32)
            for l in range(L):
                e = j * L + l
                etu = _lane_take(etv, jnp.full((L,), l, i32))
                col = etu * H
                v = ai_r.at[e][...] + aj_r.at[e][...]
                exv = jnp.exp(jnp.maximum(v, NEG * v))
                exr_r.at[e][...] = jnp.where(landc == col, exv, 0.0)
                m = lanes == l
                acc0 = jnp.where(m, _lane_take(exv, col), acc0)
                acc1 = jnp.where(m, _lane_take(exv, col + 1), acc1)
            exp0_v[pl.ds(j * L, L)] = acc0
            exp1_v[pl.ds(j * L, L)] = acc1
            return _
        lax.fori_loop(0, BA // L, exloop, None)
        pltpu.sync_copy(exr_r, den_sh.at[dst_v], add=True)
        pltpu.sync_copy(exp0_v, exp_h.at[0].at[pl.ds(base, BA)])
        pltpu.sync_copy(exp1_v, exp_h.at[1].at[pl.ds(base, BA)])
        pltpu.sync_copy(key_v, key_h.at[pl.ds(base, BA)])
        return _
    lax.fori_loop(0, (NBLK - wid + NW - 1) // NW, block, None)

    plsc.subcore_barrier()
    pltpu.sync_copy(den_sh.at[pl.ds(t * TRD, TRD)],
                    denp_h.at[c].at[pl.ds(t * TRD, TRD)])


def _make_attn():
    return pl.kernel(
        _attn_body,
        out_type=(jax.ShapeDtypeStruct((2, E), f32),
                  jax.ShapeDtypeStruct((E,), i32),
                  jax.ShapeDtypeStruct((NC, NP, 16), f32)),
        mesh=_mesh(),
        compiler_params=pltpu.CompilerParams(use_tc_tiling_on_sc=False),
        scratch_types=[
            pltpu.VMEM_SHARED((NP, 16), f32),
            pltpu.VMEM((BA,), i32), pltpu.VMEM((BA,), i32),
            pltpu.VMEM((BA,), i32), pltpu.VMEM((BA,), i32),
            pltpu.VMEM((BA, 16), f32), pltpu.VMEM((BA, 16), f32),
            pltpu.VMEM((BA, 16), f32),
            pltpu.VMEM((BA,), f32), pltpu.VMEM((BA,), f32),
            pltpu.SemaphoreType.DMA, pltpu.SemaphoreType.DMA,
        ])


BB = 256
HB = BB // 2
NBLKB = E // BB


def _msg_body(key_h, src_h, exp_h, hj0_h, hj1_h, hj2_h, hj3_h,
              zp_h,
              z_sh, hj_sh, key_v, src_v, exm_v, hj_r, s1, s2, s3, s4):
    c = lax.axis_index("c")
    t = lax.axis_index("s")
    hj_tabs = (hj0_h, hj1_h, hj2_h, hj3_h)
    lanes = lax.iota(i32, L)

    for cc in range(4):
        h = cc // 2

        @pl.when(c == h)
        def _chunk():
            def zrow(i, _):
                hj_r.at[i][pl.ds(0, L)] = jnp.zeros((L,), f32)
                hj_r.at[i][pl.ds(L, L)] = jnp.zeros((L,), f32)
                return _
            lax.fori_loop(0, BB, zrow, None)
            for k in range(12):
                pltpu.sync_copy(hj_r, z_sh.at[pl.ds(t * TRK + k * BB, BB)])
            pltpu.sync_copy(hj_r.at[pl.ds(0, 56)],
                            z_sh.at[pl.ds(t * TRK + 12 * BB, 56)])
            pltpu.sync_copy(hj_tabs[cc].at[pl.ds(t * TRD, TRD)],
                            hj_sh.at[pl.ds(t * TRD, TRD)])
            plsc.subcore_barrier()

            def block(b, _):
                base = (b * NS + t) * BB
                pltpu.sync_copy(key_h.at[pl.ds(base, BB)], key_v)
                pltpu.sync_copy(src_h.at[pl.ds(base, BB)], src_v)
                cp1 = pltpu.async_copy(exp_h.at[h].at[pl.ds(base, BB)],
                                       exm_v, s1)
                g0 = pltpu.async_copy(hj_sh.at[src_v.at[pl.ds(0, HB)]],
                                      hj_r.at[pl.ds(0, HB)], s2)
                g1 = pltpu.async_copy(hj_sh.at[src_v.at[pl.ds(HB, HB)]],
                                      hj_r.at[pl.ds(HB, HB)], s3)
                cp1.wait()
                g0.wait()

                def scale(j, _):
                    row = exm_v[pl.ds(j * L, L)]
                    for l in range(L):
                        e = j * L + l
                        sv = _lane_take(row, jnp.full((L,), l, i32))
                        r0 = hj_r.at[e][pl.ds(0, L)]
                        r1 = hj_r.at[e][pl.ds(L, L)]
                        hj_r.at[e][pl.ds(0, L)] = r0 * sv
                        hj_r.at[e][pl.ds(L, L)] = r1 * sv
                    return _
                lax.fori_loop(0, HB // L, scale, None)
                sc0 = pltpu.async_copy(hj_r.at[pl.ds(0, HB)],
                                       z_sh.at[key_v.at[pl.ds(0, HB)]],
                                       s4, add=True)
                g1.wait()
                lax.fori_loop(HB // L, BB // L, scale, None)
                pltpu.sync_copy(hj_r.at[pl.ds(HB, HB)],
                                z_sh.at[key_v.at[pl.ds(HB, HB)]], add=True)
                sc0.wait()
                return _
            lax.fori_loop(0, (NBLKB - t + NS - 1) // NS, block, None)

            plsc.subcore_barrier()
            pltpu.sync_copy(z_sh.at[pl.ds(t * TRK, TRK)],
                            zp_h.at[cc].at[pl.ds(t * TRK, TRK)])
            plsc.subcore_barrier()


def _make_msg():
    return pl.kernel(
        _msg_body,
        out_type=jax.ShapeDtypeStruct((4, KS, 32), f32),
        mesh=_mesh(),
        compiler_params=pltpu.CompilerParams(use_tc_tiling_on_sc=False),
        scratch_types=[
            pltpu.VMEM_SHARED((KS, 32), f32),
            pltpu.VMEM_SHARED((NP, 32), f32),
            pltpu.VMEM((BB,), i32), pltpu.VMEM((BB,), i32),
            pltpu.VMEM((BB,), f32), pltpu.VMEM((BB, 32), f32),
            pltpu.SemaphoreType.DMA, pltpu.SemaphoreType.DMA,
            pltpu.SemaphoreType.DMA, pltpu.SemaphoreType.DMA,
        ])


def _pre_tc(x, wcat):
    M, K = x.shape
    bn = 1024
    grid = (M // bn,)

    def body(x_ref, w_ref, hj0, hj1, hj2, hj3, sn, ai, aj, xs):
        y = jnp.dot(x_ref[...], w_ref[...], preferred_element_type=f32)
        hj0[...] = y[:, 0:32]
        hj1[...] = y[:, 32:64]
        hj2[...] = y[:, 64:96]
        hj3[...] = y[:, 96:128]
        sn[...] = y[:, 128:256]
        ai[...] = y[:, 256:272]
        aj[...] = y[:, 272:288]
        xs[...] = y[:, 288:352]

    outs = [jax.ShapeDtypeStruct((M, 32), f32)] * 4 + [
        jax.ShapeDtypeStruct((M, 128), f32),
        jax.ShapeDtypeStruct((M, 16), f32),
        jax.ShapeDtypeStruct((M, 16), f32),
        jax.ShapeDtypeStruct((M, 64), f32),
    ]
    ospec = [pl.BlockSpec((bn, s.shape[1]), lambda i: (i, 0)) for s in outs]
    return pl.pallas_call(
        body,
        grid=grid,
        in_specs=[pl.BlockSpec((bn, K), lambda i: (i, 0)),
                  pl.BlockSpec((K, 352), lambda i: (0, 0))],
        out_specs=ospec,
        out_shape=outs,
    )(x, wcat)


def _post_tc(zp, denp, sn, xs, wqkv, wrel, last):
    bn = 400
    grid = (N // bn,)

    def body(z0, z1, z2, z3, z4, dn_r, sn_r, xs_r, wq_r, wr_r, out):
        zrefs = (z0, z1, z2, z3, z4)
        den = dn_r[0] + dn_r[1]
        den = jnp.where(den > 0, den, 1.0)
        sn_b = sn_r[...]
        xs_b = xs_r[...]
        qs, ks, vs = [], [], []
        for r in range(R):
            zr = zrefs[r][...]
            d0 = den[:, 2 * r:2 * r + 1]
            d1 = den[:, 2 * r + 1:2 * r + 2]
            zcat = jnp.concatenate(
                [zr[0] / d0, zr[1] / d0, zr[2] / d1, zr[3] / d1], axis=1)
            zfull = zcat + sn_b
            qkv = jnp.dot(zfull, wq_r[r], preferred_element_type=f32)
            qs.append(qkv[:, 0:64])
            ks.append(qkv[:, 64:128])
            vs.append(qkv[:, 128:192])
        acc = jnp.zeros((bn, 64), f32)
        for r in range(R):
            es = [jnp.exp(jnp.sum(qs[r] * ks[s2], axis=1, keepdims=True))
                  for s2 in range(R)]
            tot = es[0] + es[1] + es[2] + es[3] + es[4]
            delta = sum((es[s2] / tot) * vs[s2] for s2 in range(R))
            m = (jnp.sum(delta, axis=1, keepdims=True) != 0).astype(f32)
            acc = acc + (delta + xs_b * m) * wr_r[r:r + 1, 0:1]
        if last:
            mx = jnp.max(acc, axis=1, keepdims=True)
            lse = mx + jnp.log(jnp.sum(jnp.exp(acc - mx), axis=1,
                                       keepdims=True))
            out[...] = acc - lse
        else:
            out[...] = jnp.maximum(acc, 0.0)

    def zspec(r):
        return pl.BlockSpec((4, bn, 32),
                            lambda i, r=r: (0, r * (N // bn) + i, 0))

    return pl.pallas_call(
        body,
        grid=grid,
        in_specs=[zspec(0), zspec(1), zspec(2), zspec(3), zspec(4),
                  pl.BlockSpec((NC, bn, 16), lambda i: (0, i, 0)),
                  pl.BlockSpec((bn, 128), lambda i: (i, 0)),
                  pl.BlockSpec((bn, 64), lambda i: (i, 0)),
                  pl.BlockSpec((R, 128, 192), lambda i: (0, 0, 0)),
                  pl.BlockSpec((8, 128), lambda i: (0, 0))],
        out_specs=pl.BlockSpec((bn, 64), lambda i: (i, 0)),
        out_shape=jax.ShapeDtypeStruct((N, 64), f32),
    )(zp, zp, zp, zp, zp, denp, sn, xs, wqkv, wrel)


def _fold_weights(p, out_c):
    att = p['node_att']
    att_i = att[:, :, :out_c]
    att_j = att[:, :, out_c:]
    eye = jnp.eye(H, dtype=f32)
    Mi = jnp.einsum('rhc,hk->hcrk', att_i, eye).reshape(H * out_c, R * H)
    Mj = jnp.einsum('rhc,hk->hcrk', att_j, eye).reshape(H * out_c, R * H)
    pad = jnp.zeros((p['lin_i'].shape[0], 16 - R * H), f32)
    w_ai = jnp.concatenate([p['lin_i'] @ Mi, pad], axis=1)
    w_aj = jnp.concatenate([p['lin_j'] @ Mj, pad], axis=1)
    wcat = jnp.concatenate(
        [p['lin_j'], p['W_self_node'], w_ai, w_aj, p['W_self']], axis=1)
    wqkv = jnp.concatenate([p['W_q'], p['W_k'], p['W_v']], axis=2)
    wrel = jnp.zeros((8, 128), f32).at[:R, 0].set(p['W_relation'][:, 0])
    return wcat, wqkv, wrel


_attn_call = _make_attn()
_msg_call = _make_msg()


def kernel(n_id, local_node_idx, edge_index, edge_type, node_type, emb,
           params):
    src = edge_index[0]
    dst = edge_index[1]
    wcat1, wqkv1, wrel1 = _fold_weights(params[0], 64)
    wcat2, wqkv2, wrel2 = _fold_weights(params[1], 64)

    idxp = jnp.concatenate([local_node_idx, jnp.zeros((NP - N,), i32)])
    x = _gather_rows(emb, idxp)

    for li, (wcat, wqkv, wrel) in enumerate(
            [(wcat1, wqkv1, wrel1), (wcat2, wqkv2, wrel2)]):
        hj0, hj1, hj2, hj3, sn, ai, aj, xs = _pre_tc(x, wcat)
        exr, key, denp = _attn_call(src, dst, edge_type, ai, aj)
        zp = _msg_call(key, src, exr, hj0, hj1, hj2, hj3)
        zp = zp[:, :R * N, :]
        x = _post_tc(zp, denp, sn, xs, wqkv, wrel, last=(li == 1))
        if li == 0:
            x = jnp.concatenate([x, jnp.zeros((NP - N, 64), f32)], axis=0)
    return x


# speedup vs baseline: 18.9643x; 1.1315x over previous
import functools
import jax
import jax.numpy as jnp
from jax import lax
from jax.experimental import pallas as pl
from jax.experimental.pallas import tpu as pltpu
from jax.experimental.pallas import tpu_sc as plsc

N = 10000
E = 320000
R = 5
H = 2
NEG = 0.2

NC, NS, L = 2, 16, 16
NW = NC * NS
NP = 10240
KS = 50048
TRK = KS // NS
TRD = NP // NS

f32 = jnp.float32
i32 = jnp.int32


def _mesh():
    return plsc.VectorSubcoreMesh(
        core_axis_name="c", subcore_axis_name="s",
        num_cores=NC, num_subcores=NS)


def _lane_take(v, idx):
    return lax.gather(
        v, idx[:, None],
        lax.GatherDimensionNumbers(offset_dims=(), collapsed_slice_dims=(0,),
                                   start_index_map=(0,)),
        (1,), mode=lax.GatherScatterMode.PROMISE_IN_BOUNDS)


@functools.partial(
    pl.kernel, mesh=_mesh(),
    compiler_params=pltpu.CompilerParams(use_tc_tiling_on_sc=False),
    out_type=jax.ShapeDtypeStruct((NP, 128), f32),
    scratch_types=[
        pltpu.VMEM((128,), i32),
        pltpu.VMEM((128, 128), f32),
        pltpu.SemaphoreType.DMA,
    ])
def _gather_rows(emb_hbm, idx_hbm, out_hbm, idx_v, rows_v, sem):
    wid = lax.axis_index("s") * NC + lax.axis_index("c")
    nb = (NP // 128 - wid + NW - 1) // NW

    def blk(b, _):
        base = (b * NW + wid) * 128
        pltpu.sync_copy(idx_hbm.at[pl.ds(base, 128)], idx_v)
        pltpu.async_copy(emb_hbm.at[idx_v], rows_v, sem).wait()
        pltpu.sync_copy(rows_v, out_hbm.at[pl.ds(base, 128)])
        return _
    lax.fori_loop(0, nb, blk, None)


BA = 512
NBLK = E // BA


def _attn_body(src_h, dst_h, et_h, ai_h, aj_h,
               exp_h, key_h, denp_h,
               den_sh, src_v, dst_v, et_v, key_v,
               ai_r, aj_r, exr_r, exp0_v, exp1_v, s1, s2):
    c = lax.axis_index("c")
    t = lax.axis_index("s")
    lanes = lax.iota(i32, L)

    def zrow(i, _):
        exr_r.at[i][...] = jnp.zeros((L,), f32)
        return _
    lax.fori_loop(0, BA, zrow, None)
    pltpu.sync_copy(exr_r, den_sh.at[pl.ds(t * TRD, BA)])
    pltpu.sync_copy(exr_r.at[pl.ds(0, TRD - BA)],
                    den_sh.at[pl.ds(t * TRD + BA, TRD - BA)])
    plsc.subcore_barrier()

    wid = c * NS + t

    def block(b, _):
        base = (b * NW + wid) * BA
        pltpu.sync_copy(src_h.at[pl.ds(base, BA)], src_v)
        pltpu.sync_copy(dst_h.at[pl.ds(base, BA)], dst_v)
        pltpu.sync_copy(et_h.at[pl.ds(base, BA)], et_v)

        def keyloop(j, _):
            sl = pl.ds(j * L, L)
            key_v[sl] = et_v[sl] * N + dst_v[sl]
            return _
        lax.fori_loop(0, BA // L, keyloop, None)
        cp1 = pltpu.async_copy(ai_h.at[dst_v], ai_r, s1)
        cp2 = pltpu.async_copy(aj_h.at[src_v], aj_r, s2)
        cp1.wait()
        cp2.wait()

        landc = lanes & -2

        def exloop(j, _):
            etv = et_v[pl.ds(j * L, L)]
            acc0 = jnp.zeros((L,), f32)
            acc1 = jnp.zeros((L,), f32)
            for l in range(L):
                e = j * L + l
                etu = _lane_take(etv, jnp.full((L,), l, i32))
                col = etu * H
                v = ai_r.at[e][...] + aj_r.at[e][...]
                exv = jnp.exp(jnp.maximum(v, NEG * v))
                exr_r.at[e][...] = jnp.where(landc == col, exv, 0.0)
                m = lanes == l
                acc0 = jnp.where(m, _lane_take(exv, col), acc0)
                acc1 = jnp.where(m, _lane_take(exv, col + 1), acc1)
            exp0_v[pl.ds(j * L, L)] = acc0
            exp1_v[pl.ds(j * L, L)] = acc1
            return _
        lax.fori_loop(0, BA // L, exloop, None)
        pltpu.sync_copy(exr_r, den_sh.at[dst_v], add=True)
        pltpu.sync_copy(exp0_v, exp_h.at[0].at[pl.ds(base, BA)])
        pltpu.sync_copy(exp1_v, exp_h.at[1].at[pl.ds(base, BA)])
        pltpu.sync_copy(key_v, key_h.at[pl.ds(base, BA)])
        return _
    lax.fori_loop(0, (NBLK - wid + NW - 1) // NW, block, None)

    plsc.subcore_barrier()
    pltpu.sync_copy(den_sh.at[pl.ds(t * TRD, TRD)],
                    denp_h.at[c].at[pl.ds(t * TRD, TRD)])


def _make_attn():
    return pl.kernel(
        _attn_body,
        out_type=(jax.ShapeDtypeStruct((2, E), f32),
                  jax.ShapeDtypeStruct((E,), i32),
                  jax.ShapeDtypeStruct((NC, NP, 16), f32)),
        mesh=_mesh(),
        compiler_params=pltpu.CompilerParams(use_tc_tiling_on_sc=False),
        scratch_types=[
            pltpu.VMEM_SHARED((NP, 16), f32),
            pltpu.VMEM((BA,), i32), pltpu.VMEM((BA,), i32),
            pltpu.VMEM((BA,), i32), pltpu.VMEM((BA,), i32),
            pltpu.VMEM((BA, 16), f32), pltpu.VMEM((BA, 16), f32),
            pltpu.VMEM((BA, 16), f32),
            pltpu.VMEM((BA,), f32), pltpu.VMEM((BA,), f32),
            pltpu.SemaphoreType.DMA, pltpu.SemaphoreType.DMA,
        ])


BB = 256
HB = BB // 2
NBLKB = E // BB


def _msg_body(key_h, src_h, exp_h, hj0_h, hj1_h, hj2_h, hj3_h,
              zp_h,
              z_sh, hj_sh, key_v, src_v, exm_v, hj_r, s1, s2, s3, s4):
    c = lax.axis_index("c")
    t = lax.axis_index("s")
    hj_tabs = (hj0_h, hj1_h, hj2_h, hj3_h)
    lanes = lax.iota(i32, L)

    for cc in range(4):
        h = cc // 2

        @pl.when(c == h)
        def _chunk():
            def zrow(i, _):
                hj_r.at[i][pl.ds(0, L)] = jnp.zeros((L,), f32)
                hj_r.at[i][pl.ds(L, L)] = jnp.zeros((L,), f32)
                return _
            lax.fori_loop(0, BB, zrow, None)
            for k in range(12):
                pltpu.sync_copy(hj_r, z_sh.at[pl.ds(t * TRK + k * BB, BB)])
            pltpu.sync_copy(hj_r.at[pl.ds(0, 56)],
                            z_sh.at[pl.ds(t * TRK + 12 * BB, 56)])
            pltpu.sync_copy(hj_tabs[cc].at[pl.ds(t * TRD, TRD)],
                            hj_sh.at[pl.ds(t * TRD, TRD)])
            plsc.subcore_barrier()

            def block(b, _):
                base = (b * NS + t) * BB
                pltpu.sync_copy(key_h.at[pl.ds(base, BB)], key_v)
                pltpu.sync_copy(src_h.at[pl.ds(base, BB)], src_v)
                cp1 = pltpu.async_copy(exp_h.at[h].at[pl.ds(base, BB)],
                                       exm_v, s1)
                g0 = pltpu.async_copy(hj_sh.at[src_v.at[pl.ds(0, HB)]],
                                      hj_r.at[pl.ds(0, HB)], s2)
                g1 = pltpu.async_copy(hj_sh.at[src_v.at[pl.ds(HB, HB)]],
                                      hj_r.at[pl.ds(HB, HB)], s3)
                cp1.wait()
                g0.wait()

                def scale(j, _):
                    row = exm_v[pl.ds(j * L, L)]
                    for l in range(L):
                        e = j * L + l
                        sv = _lane_take(row, jnp.full((L,), l, i32))
                        r0 = hj_r.at[e][pl.ds(0, L)]
                        r1 = hj_r.at[e][pl.ds(L, L)]
                        hj_r.at[e][pl.ds(0, L)] = r0 * sv
                        hj_r.at[e][pl.ds(L, L)] = r1 * sv
                    return _
                lax.fori_loop(0, HB // L, scale, None)
                sc0 = pltpu.async_copy(hj_r.at[pl.ds(0, HB)],
                                       z_sh.at[key_v.at[pl.ds(0, HB)]],
                                       s4, add=True)
                g1.wait()
                lax.fori_loop(HB // L, BB // L, scale, None)
                pltpu.sync_copy(hj_r.at[pl.ds(HB, HB)],
                                z_sh.at[key_v.at[pl.ds(HB, HB)]], add=True)
                sc0.wait()
                return _
            lax.fori_loop(0, (NBLKB - t + NS - 1) // NS, block, None)

            plsc.subcore_barrier()
            pltpu.sync_copy(z_sh.at[pl.ds(t * TRK, TRK)],
                            zp_h.at[pl.ds(t * TRK, TRK), pl.ds(cc * 32, 32)])
            plsc.subcore_barrier()


def _make_msg():
    return pl.kernel(
        _msg_body,
        out_type=jax.ShapeDtypeStruct((KS, 128), f32),
        mesh=_mesh(),
        compiler_params=pltpu.CompilerParams(use_tc_tiling_on_sc=False),
        scratch_types=[
            pltpu.VMEM_SHARED((KS, 32), f32),
            pltpu.VMEM_SHARED((NP, 32), f32),
            pltpu.VMEM((BB,), i32), pltpu.VMEM((BB,), i32),
            pltpu.VMEM((BB,), f32), pltpu.VMEM((BB, 32), f32),
            pltpu.SemaphoreType.DMA, pltpu.SemaphoreType.DMA,
            pltpu.SemaphoreType.DMA, pltpu.SemaphoreType.DMA,
        ])


def _pre_tc(x, wcat):
    M, K = x.shape
    bn = 1024
    grid = (M // bn,)

    def body(x_ref, w_ref, hj0, hj1, hj2, hj3, sn, ai, aj, xs):
        y = jnp.dot(x_ref[...], w_ref[...], preferred_element_type=f32)
        hj0[...] = y[:, 0:32]
        hj1[...] = y[:, 32:64]
        hj2[...] = y[:, 64:96]
        hj3[...] = y[:, 96:128]
        sn[...] = y[:, 128:256]
        ai[...] = y[:, 256:272]
        aj[...] = y[:, 272:288]
        xs[...] = y[:, 288:352]

    outs = [jax.ShapeDtypeStruct((M, 32), f32)] * 4 + [
        jax.ShapeDtypeStruct((M, 128), f32),
        jax.ShapeDtypeStruct((M, 16), f32),
        jax.ShapeDtypeStruct((M, 16), f32),
        jax.ShapeDtypeStruct((M, 64), f32),
    ]
    ospec = [pl.BlockSpec((bn, s.shape[1]), lambda i: (i, 0)) for s in outs]
    return pl.pallas_call(
        body,
        grid=grid,
        in_specs=[pl.BlockSpec((bn, K), lambda i: (i, 0)),
                  pl.BlockSpec((K, 352), lambda i: (0, 0))],
        out_specs=ospec,
        out_shape=outs,
    )(x, wcat)


def _post_tc(zp, denp, sn, xs, wqkv, wrel, last):
    bn = 400
    grid = (N // bn,)
    bsel = (lax.broadcasted_iota(i32, (R * R * 64, 128), 0) // 64 ==
            lax.broadcasted_iota(i32, (R * R * 64, 128), 1)).astype(f32)
    bsum = (lax.broadcasted_iota(i32, (R * 64, 128), 0) // 64 ==
            lax.broadcasted_iota(i32, (R * 64, 128), 1)).astype(f32)

    def body(z0, z1, z2, z3, z4, dn_r, sn_r, xs_r, wq_r, wr_r,
             bsel_r, bsum_r, out):
        zrefs = (z0, z1, z2, z3, z4)
        den = dn_r[0] + dn_r[1]
        den = jnp.where(den > 0, den, 1.0)
        dinv = 1.0 / den
        sn_b = sn_r[...]
        xs_b = xs_r[...]
        qs, ks, vs = [], [], []
        for r in range(R):
            zr = zrefs[r][...]
            d0 = jnp.broadcast_to(dinv[:, 2 * r:2 * r + 1], (bn, 64))
            d1 = jnp.broadcast_to(dinv[:, 2 * r + 1:2 * r + 2], (bn, 64))
            zfull = zr * jnp.concatenate([d0, d1], axis=1) + sn_b
            qkv = jnp.dot(zfull, wq_r[r], preferred_element_type=f32)
            qs.append(qkv[:, 0:64])
            ks.append(qkv[:, 64:128])
            vs.append(qkv[:, 128:192])
        m_cat = jnp.concatenate(
            [qs[r] * ks[s] for r in range(R) for s in range(R)], axis=1)
        es = jnp.exp(jnp.dot(m_cat, bsel_r[...],
                             preferred_element_type=f32))
        deltas = []
        for r in range(R):
            tot = (es[:, 5 * r:5 * r + 1] + es[:, 5 * r + 1:5 * r + 2]
                   + es[:, 5 * r + 2:5 * r + 3] + es[:, 5 * r + 3:5 * r + 4]
                   + es[:, 5 * r + 4:5 * r + 5])
            ti = 1.0 / tot
            deltas.append(sum((es[:, 5 * r + s:5 * r + s + 1] * ti) * vs[s]
                              for s in range(R)))
        d_cat = jnp.concatenate(deltas, axis=1)
        dsums = jnp.dot(d_cat, bsum_r[...], preferred_element_type=f32)
        acc = jnp.zeros((bn, 64), f32)
        for r in range(R):
            m = (dsums[:, r:r + 1] != 0).astype(f32)
            acc = acc + (deltas[r] + xs_b * m) * wr_r[r:r + 1, 0:1]
        if last:
            mx = jnp.max(acc, axis=1, keepdims=True)
            lse = mx + jnp.log(jnp.sum(jnp.exp(acc - mx), axis=1,
                                       keepdims=True))
            out[...] = acc - lse
        else:
            out[...] = jnp.maximum(acc, 0.0)

    def zspec(r):
        return pl.BlockSpec((bn, 128),
                            lambda i, r=r: (r * (N // bn) + i, 0))

    return pl.pallas_call(
        body,
        grid=grid,
        in_specs=[zspec(0), zspec(1), zspec(2), zspec(3), zspec(4),
                  pl.BlockSpec((NC, bn, 16), lambda i: (0, i, 0)),
                  pl.BlockSpec((bn, 128), lambda i: (i, 0)),
                  pl.BlockSpec((bn, 64), lambda i: (i, 0)),
                  pl.BlockSpec((R, 128, 192), lambda i: (0, 0, 0)),
                  pl.BlockSpec((8, 128), lambda i: (0, 0)),
                  pl.BlockSpec((R * R * 64, 128), lambda i: (0, 0)),
                  pl.BlockSpec((R * 64, 128), lambda i: (0, 0))],
        out_specs=pl.BlockSpec((bn, 64), lambda i: (i, 0)),
        out_shape=jax.ShapeDtypeStruct((N, 64), f32),
    )(zp, zp, zp, zp, zp, denp, sn, xs, wqkv, wrel, bsel, bsum)


def _fold_weights(p, out_c):
    att = p['node_att']
    att_i = att[:, :, :out_c]
    att_j = att[:, :, out_c:]
    eye = jnp.eye(H, dtype=f32)
    Mi = jnp.einsum('rhc,hk->hcrk', att_i, eye).reshape(H * out_c, R * H)
    Mj = jnp.einsum('rhc,hk->hcrk', att_j, eye).reshape(H * out_c, R * H)
    pad = jnp.zeros((p['lin_i'].shape[0], 16 - R * H), f32)
    w_ai = jnp.concatenate([p['lin_i'] @ Mi, pad], axis=1)
    w_aj = jnp.concatenate([p['lin_j'] @ Mj, pad], axis=1)
    wcat = jnp.concatenate(
        [p['lin_j'], p['W_self_node'], w_ai, w_aj, p['W_self']], axis=1)
    wqkv = jnp.concatenate([p['W_q'], p['W_k'], p['W_v']], axis=2)
    wrel = jnp.zeros((8, 128), f32).at[:R, 0].set(p['W_relation'][:, 0])
    return wcat, wqkv, wrel


_attn_call = _make_attn()
_msg_call = _make_msg()


def kernel(n_id, local_node_idx, edge_index, edge_type, node_type, emb,
           params):
    src = edge_index[0]
    dst = edge_index[1]
    wcat1, wqkv1, wrel1 = _fold_weights(params[0], 64)
    wcat2, wqkv2, wrel2 = _fold_weights(params[1], 64)

    idxp = jnp.concatenate([local_node_idx, jnp.zeros((NP - N,), i32)])
    x = _gather_rows(emb, idxp)

    for li, (wcat, wqkv, wrel) in enumerate(
            [(wcat1, wqkv1, wrel1), (wcat2, wqkv2, wrel2)]):
        hj0, hj1, hj2, hj3, sn, ai, aj, xs = _pre_tc(x, wcat)
        exr, key, denp = _attn_call(src, dst, edge_type, ai, aj)
        zp = _msg_call(key, src, exr, hj0, hj1, hj2, hj3)
        x = _post_tc(zp, denp, sn, xs, wqkv, wrel, last=(li == 1))
        if li == 0:
            x = jnp.concatenate([x, jnp.zeros((NP - N, 64), f32)], axis=0)
    return x


# speedup vs baseline: 19.3116x; 1.0183x over previous
import functools
import jax
import jax.numpy as jnp
from jax import lax
from jax.experimental import pallas as pl
from jax.experimental.pallas import tpu as pltpu
from jax.experimental.pallas import tpu_sc as plsc

N = 10000
E = 320000
R = 5
H = 2
NEG = 0.2

NC, NS, L = 2, 16, 16
NW = NC * NS
NP = 10240
KS = 50048
TRK = KS // NS
TRD = NP // NS

f32 = jnp.float32
i32 = jnp.int32


def _mesh():
    return plsc.VectorSubcoreMesh(
        core_axis_name="c", subcore_axis_name="s",
        num_cores=NC, num_subcores=NS)


def _lane_take(v, idx):
    return lax.gather(
        v, idx[:, None],
        lax.GatherDimensionNumbers(offset_dims=(), collapsed_slice_dims=(0,),
                                   start_index_map=(0,)),
        (1,), mode=lax.GatherScatterMode.PROMISE_IN_BOUNDS)


@functools.partial(
    pl.kernel, mesh=_mesh(),
    compiler_params=pltpu.CompilerParams(use_tc_tiling_on_sc=False),
    out_type=jax.ShapeDtypeStruct((NP, 128), f32),
    scratch_types=[
        pltpu.VMEM((128,), i32),
        pltpu.VMEM((128, 128), f32),
        pltpu.SemaphoreType.DMA,
    ])
def _gather_rows(emb_hbm, idx_hbm, out_hbm, idx_v, rows_v, sem):
    wid = lax.axis_index("s") * NC + lax.axis_index("c")
    nb = (NP // 128 - wid + NW - 1) // NW

    def blk(b, _):
        base = (b * NW + wid) * 128
        pltpu.sync_copy(idx_hbm.at[pl.ds(base, 128)], idx_v)
        pltpu.async_copy(emb_hbm.at[idx_v], rows_v, sem).wait()
        pltpu.sync_copy(rows_v, out_hbm.at[pl.ds(base, 128)])
        return _
    lax.fori_loop(0, nb, blk, None)


BA = 512
NBLK = E // BA


def _attn_body(src_h, dst_h, et_h, ai_h, aj_h,
               exp_h, key_h, denp_h,
               den_sh, src_v, dst_v, et_v, key_v,
               ai_r, aj_r, exr_r, exp0_v, exp1_v, s1, s2):
    c = lax.axis_index("c")
    t = lax.axis_index("s")
    lanes = lax.iota(i32, L)

    def zrow(i, _):
        exr_r.at[i][...] = jnp.zeros((L,), f32)
        return _
    lax.fori_loop(0, BA, zrow, None)
    pltpu.sync_copy(exr_r, den_sh.at[pl.ds(t * TRD, BA)])
    pltpu.sync_copy(exr_r.at[pl.ds(0, TRD - BA)],
                    den_sh.at[pl.ds(t * TRD + BA, TRD - BA)])
    plsc.subcore_barrier()

    wid = c * NS + t

    def block(b, _):
        base = (b * NW + wid) * BA
        pltpu.sync_copy(src_h.at[pl.ds(base, BA)], src_v)
        pltpu.sync_copy(dst_h.at[pl.ds(base, BA)], dst_v)
        pltpu.sync_copy(et_h.at[pl.ds(base, BA)], et_v)

        def keyloop(j, _):
            sl = pl.ds(j * L, L)
            key_v[sl] = et_v[sl] * N + dst_v[sl]
            return _
        lax.fori_loop(0, BA // L, keyloop, None)
        cp1 = pltpu.async_copy(ai_h.at[dst_v], ai_r, s1)
        cp2 = pltpu.async_copy(aj_h.at[src_v], aj_r, s2)
        cp1.wait()
        cp2.wait()

        landc = lanes & -2

        def exloop(j, _):
            etv = et_v[pl.ds(j * L, L)]
            acc0 = jnp.zeros((L,), f32)
            acc1 = jnp.zeros((L,), f32)
            for l in range(L):
                e = j * L + l
                etu = _lane_take(etv, jnp.full((L,), l, i32))
                col = etu * H
                v = ai_r.at[e][...] + aj_r.at[e][...]
                exv = jnp.exp(jnp.maximum(v, NEG * v))
                exr_r.at[e][...] = jnp.where(landc == col, exv, 0.0)
                m = lanes == l
                acc0 = jnp.where(m, _lane_take(exv, col), acc0)
                acc1 = jnp.where(m, _lane_take(exv, col + 1), acc1)
            exp0_v[pl.ds(j * L, L)] = acc0
            exp1_v[pl.ds(j * L, L)] = acc1
            return _
        lax.fori_loop(0, BA // L, exloop, None)
        pltpu.sync_copy(exr_r, den_sh.at[dst_v], add=True)
        pltpu.sync_copy(exp0_v, exp_h.at[0].at[pl.ds(base, BA)])
        pltpu.sync_copy(exp1_v, exp_h.at[1].at[pl.ds(base, BA)])
        pltpu.sync_copy(key_v, key_h.at[pl.ds(base, BA)])
        return _
    lax.fori_loop(0, (NBLK - wid + NW - 1) // NW, block, None)

    plsc.subcore_barrier()
    pltpu.sync_copy(den_sh.at[pl.ds(t * TRD, TRD)],
                    denp_h.at[c].at[pl.ds(t * TRD, TRD)])


def _make_attn():
    return pl.kernel(
        _attn_body,
        out_type=(jax.ShapeDtypeStruct((2, E), f32),
                  jax.ShapeDtypeStruct((E,), i32),
                  jax.ShapeDtypeStruct((NC, NP, 16), f32)),
        mesh=_mesh(),
        compiler_params=pltpu.CompilerParams(use_tc_tiling_on_sc=False),
        scratch_types=[
            pltpu.VMEM_SHARED((NP, 16), f32),
            pltpu.VMEM((BA,), i32), pltpu.VMEM((BA,), i32),
            pltpu.VMEM((BA,), i32), pltpu.VMEM((BA,), i32),
            pltpu.VMEM((BA, 16), f32), pltpu.VMEM((BA, 16), f32),
            pltpu.VMEM((BA, 16), f32),
            pltpu.VMEM((BA,), f32), pltpu.VMEM((BA,), f32),
            pltpu.SemaphoreType.DMA, pltpu.SemaphoreType.DMA,
        ])


BB = 256
HB = BB // 2
NBLKB = E // BB


def _msg_body(key_h, src_h, exp_h, hj_h,
              zp_h,
              z_sh, hj_sh, key_v, src_v, exm_v, hj_r, s1, s2, s3, s4):
    c = lax.axis_index("c")
    t = lax.axis_index("s")
    lanes = lax.iota(i32, L)

    for cc in range(4):
        h = cc // 2

        @pl.when(c == h)
        def _chunk():
            def zrow(i, _):
                hj_r.at[i][pl.ds(0, L)] = jnp.zeros((L,), f32)
                hj_r.at[i][pl.ds(L, L)] = jnp.zeros((L,), f32)
                return _
            lax.fori_loop(0, BB, zrow, None)
            for k in range(12):
                pltpu.sync_copy(hj_r, z_sh.at[pl.ds(t * TRK + k * BB, BB)])
            pltpu.sync_copy(hj_r.at[pl.ds(0, 56)],
                            z_sh.at[pl.ds(t * TRK + 12 * BB, 56)])
            pltpu.sync_copy(hj_h.at[pl.ds(t * TRD, TRD), pl.ds(cc * 32, 32)],
                            hj_sh.at[pl.ds(t * TRD, TRD)])
            plsc.subcore_barrier()

            def block(b, _):
                base = (b * NS + t) * BB
                pltpu.sync_copy(key_h.at[pl.ds(base, BB)], key_v)
                pltpu.sync_copy(src_h.at[pl.ds(base, BB)], src_v)
                cp1 = pltpu.async_copy(exp_h.at[h].at[pl.ds(base, BB)],
                                       exm_v, s1)
                g0 = pltpu.async_copy(hj_sh.at[src_v.at[pl.ds(0, HB)]],
                                      hj_r.at[pl.ds(0, HB)], s2)
                g1 = pltpu.async_copy(hj_sh.at[src_v.at[pl.ds(HB, HB)]],
                                      hj_r.at[pl.ds(HB, HB)], s3)
                cp1.wait()
                g0.wait()

                def scale(j, _):
                    row = exm_v[pl.ds(j * L, L)]
                    for l in range(L):
                        e = j * L + l
                        sv = _lane_take(row, jnp.full((L,), l, i32))
                        r0 = hj_r.at[e][pl.ds(0, L)]
                        r1 = hj_r.at[e][pl.ds(L, L)]
                        hj_r.at[e][pl.ds(0, L)] = r0 * sv
                        hj_r.at[e][pl.ds(L, L)] = r1 * sv
                    return _
                lax.fori_loop(0, HB // L, scale, None)
                sc0 = pltpu.async_copy(hj_r.at[pl.ds(0, HB)],
                                       z_sh.at[key_v.at[pl.ds(0, HB)]],
                                       s4, add=True)
                g1.wait()
                lax.fori_loop(HB // L, BB // L, scale, None)
                pltpu.sync_copy(hj_r.at[pl.ds(HB, HB)],
                                z_sh.at[key_v.at[pl.ds(HB, HB)]], add=True)
                sc0.wait()
                return _
            lax.fori_loop(0, (NBLKB - t + NS - 1) // NS, block, None)

            plsc.subcore_barrier()
            pltpu.sync_copy(z_sh.at[pl.ds(t * TRK, TRK)],
                            zp_h.at[pl.ds(t * TRK, TRK), pl.ds(cc * 32, 32)])
            plsc.subcore_barrier()


def _make_msg():
    return pl.kernel(
        _msg_body,
        out_type=jax.ShapeDtypeStruct((KS, 128), f32),
        mesh=_mesh(),
        compiler_params=pltpu.CompilerParams(use_tc_tiling_on_sc=False),
        scratch_types=[
            pltpu.VMEM_SHARED((KS, 32), f32),
            pltpu.VMEM_SHARED((NP, 32), f32),
            pltpu.VMEM((BB,), i32), pltpu.VMEM((BB,), i32),
            pltpu.VMEM((BB,), f32), pltpu.VMEM((BB, 32), f32),
            pltpu.SemaphoreType.DMA, pltpu.SemaphoreType.DMA,
            pltpu.SemaphoreType.DMA, pltpu.SemaphoreType.DMA,
        ])


def _pre_tc(x, wcat):
    M, K = x.shape
    bn = 1024
    grid = (M // bn,)

    def body(x_ref, w_ref, hj, sn, ai, aj, xs):
        y = jnp.dot(x_ref[...], w_ref[...], preferred_element_type=f32)
        hj[...] = y[:, 0:128]
        sn[...] = y[:, 128:256]
        ai[...] = y[:, 256:272]
        aj[...] = y[:, 272:288]
        xs[...] = y[:, 288:352]

    outs = [
        jax.ShapeDtypeStruct((M, 128), f32),
        jax.ShapeDtypeStruct((M, 128), f32),
        jax.ShapeDtypeStruct((M, 16), f32),
        jax.ShapeDtypeStruct((M, 16), f32),
        jax.ShapeDtypeStruct((M, 64), f32),
    ]
    ospec = [pl.BlockSpec((bn, s.shape[1]), lambda i: (i, 0)) for s in outs]
    return pl.pallas_call(
        body,
        grid=grid,
        in_specs=[pl.BlockSpec((bn, K), lambda i: (i, 0)),
                  pl.BlockSpec((K, 352), lambda i: (0, 0))],
        out_specs=ospec,
        out_shape=outs,
    )(x, wcat)


def _post_tc(zp, denp, sn, xs, wqkv, wrel, last):
    bn = 1000
    grid = (N // bn,)
    bsel = (lax.broadcasted_iota(i32, (R * R * 64, 128), 0) // 64 ==
            lax.broadcasted_iota(i32, (R * R * 64, 128), 1)).astype(f32)
    bsum = (lax.broadcasted_iota(i32, (R * 64, 128), 0) // 64 ==
            lax.broadcasted_iota(i32, (R * 64, 128), 1)).astype(f32)

    def body(z0, z1, z2, z3, z4, dn_r, sn_r, xs_r, wq_r, wr_r,
             bsel_r, bsum_r, out):
        zrefs = (z0, z1, z2, z3, z4)
        den = dn_r[0] + dn_r[1]
        den = jnp.where(den > 0, den, 1.0)
        dinv = 1.0 / den
        sn_b = sn_r[...]
        xs_b = xs_r[...]
        qs, ks, vs = [], [], []
        for r in range(R):
            zr = zrefs[r][...]
            d0 = jnp.broadcast_to(dinv[:, 2 * r:2 * r + 1], (bn, 64))
            d1 = jnp.broadcast_to(dinv[:, 2 * r + 1:2 * r + 2], (bn, 64))
            zfull = zr * jnp.concatenate([d0, d1], axis=1) + sn_b
            qkv = jnp.dot(zfull, wq_r[r], preferred_element_type=f32)
            qs.append(qkv[:, 0:64])
            ks.append(qkv[:, 64:128])
            vs.append(qkv[:, 128:192])
        m_cat = jnp.concatenate(
            [qs[r] * ks[s] for r in range(R) for s in range(R)], axis=1)
        es = jnp.exp(jnp.dot(m_cat, bsel_r[...],
                             preferred_element_type=f32))
        deltas = []
        for r in range(R):
            tot = (es[:, 5 * r:5 * r + 1] + es[:, 5 * r + 1:5 * r + 2]
                   + es[:, 5 * r + 2:5 * r + 3] + es[:, 5 * r + 3:5 * r + 4]
                   + es[:, 5 * r + 4:5 * r + 5])
            ti = 1.0 / tot
            deltas.append(sum((es[:, 5 * r + s:5 * r + s + 1] * ti) * vs[s]
                              for s in range(R)))
        d_cat = jnp.concatenate(deltas, axis=1)
        dsums = jnp.dot(d_cat, bsum_r[...], preferred_element_type=f32)
        acc = jnp.zeros((bn, 64), f32)
        for r in range(R):
            m = (dsums[:, r:r + 1] != 0).astype(f32)
            acc = acc + (deltas[r] + xs_b * m) * wr_r[r:r + 1, 0:1]
        if last:
            mx = jnp.max(acc, axis=1, keepdims=True)
            lse = mx + jnp.log(jnp.sum(jnp.exp(acc - mx), axis=1,
                                       keepdims=True))
            out[...] = acc - lse
        else:
            out[...] = jnp.maximum(acc, 0.0)

    def zspec(r):
        return pl.BlockSpec((bn, 128),
                            lambda i, r=r: (r * (N // bn) + i, 0))

    return pl.pallas_call(
        body,
        grid=grid,
        in_specs=[zspec(0), zspec(1), zspec(2), zspec(3), zspec(4),
                  pl.BlockSpec((NC, bn, 16), lambda i: (0, i, 0)),
                  pl.BlockSpec((bn, 128), lambda i: (i, 0)),
                  pl.BlockSpec((bn, 64), lambda i: (i, 0)),
                  pl.BlockSpec((R, 128, 192), lambda i: (0, 0, 0)),
                  pl.BlockSpec((8, 128), lambda i: (0, 0)),
                  pl.BlockSpec((R * R * 64, 128), lambda i: (0, 0)),
                  pl.BlockSpec((R * 64, 128), lambda i: (0, 0))],
        out_specs=pl.BlockSpec((bn, 64), lambda i: (i, 0)),
        out_shape=jax.ShapeDtypeStruct((N, 64), f32),
    )(zp, zp, zp, zp, zp, denp, sn, xs, wqkv, wrel, bsel, bsum)


def _fold_weights(p, out_c):
    att = p['node_att']
    att_i = att[:, :, :out_c]
    att_j = att[:, :, out_c:]
    eye = jnp.eye(H, dtype=f32)
    Mi = jnp.einsum('rhc,hk->hcrk', att_i, eye).reshape(H * out_c, R * H)
    Mj = jnp.einsum('rhc,hk->hcrk', att_j, eye).reshape(H * out_c, R * H)
    pad = jnp.zeros((p['lin_i'].shape[0], 16 - R * H), f32)
    w_ai = jnp.concatenate([p['lin_i'] @ Mi, pad], axis=1)
    w_aj = jnp.concatenate([p['lin_j'] @ Mj, pad], axis=1)
    wcat = jnp.concatenate(
        [p['lin_j'], p['W_self_node'], w_ai, w_aj, p['W_self']], axis=1)
    wqkv = jnp.concatenate([p['W_q'], p['W_k'], p['W_v']], axis=2)
    wrel = jnp.zeros((8, 128), f32).at[:R, 0].set(p['W_relation'][:, 0])
    return wcat, wqkv, wrel


_attn_call = _make_attn()
_msg_call = _make_msg()


def kernel(n_id, local_node_idx, edge_index, edge_type, node_type, emb,
           params):
    src = edge_index[0]
    dst = edge_index[1]
    wcat1, wqkv1, wrel1 = _fold_weights(params[0], 64)
    wcat2, wqkv2, wrel2 = _fold_weights(params[1], 64)

    idxp = jnp.concatenate([local_node_idx, jnp.zeros((NP - N,), i32)])
    x = _gather_rows(emb, idxp)

    for li, (wcat, wqkv, wrel) in enumerate(
            [(wcat1, wqkv1, wrel1), (wcat2, wqkv2, wrel2)]):
        hj, sn, ai, aj, xs = _pre_tc(x, wcat)
        exr, key, denp = _attn_call(src, dst, edge_type, ai, aj)
        zp = _msg_call(key, src, exr, hj)
        x = _post_tc(zp, denp, sn, xs, wqkv, wrel, last=(li == 1))
        if li == 0:
            x = jnp.concatenate([x, jnp.zeros((NP - N, 64), f32)], axis=0)
    return x


# speedup vs baseline: 21.6021x; 1.1186x over previous
import functools
import jax
import jax.numpy as jnp
from jax import lax
from jax.experimental import pallas as pl
from jax.experimental.pallas import tpu as pltpu
from jax.experimental.pallas import tpu_sc as plsc

N = 10000
E = 320000
R = 5
H = 2
NEG = 0.2

NC, NS, L = 2, 16, 16
NW = NC * NS
NP = 10240
KS = 50048
TRK = KS // NS
TRD = NP // NS

f32 = jnp.float32
i32 = jnp.int32


def _mesh():
    return plsc.VectorSubcoreMesh(
        core_axis_name="c", subcore_axis_name="s",
        num_cores=NC, num_subcores=NS)


def _lane_take(v, idx):
    return lax.gather(
        v, idx[:, None],
        lax.GatherDimensionNumbers(offset_dims=(), collapsed_slice_dims=(0,),
                                   start_index_map=(0,)),
        (1,), mode=lax.GatherScatterMode.PROMISE_IN_BOUNDS)


@functools.partial(
    pl.kernel, mesh=_mesh(),
    compiler_params=pltpu.CompilerParams(use_tc_tiling_on_sc=False),
    out_type=jax.ShapeDtypeStruct((NP, 128), f32),
    scratch_types=[
        pltpu.VMEM((128,), i32),
        pltpu.VMEM((128, 128), f32),
        pltpu.SemaphoreType.DMA,
    ])
def _gather_rows(emb_hbm, idx_hbm, out_hbm, idx_v, rows_v, sem):
    wid = lax.axis_index("s") * NC + lax.axis_index("c")
    nb = (NP // 128 - wid + NW - 1) // NW

    def blk(b, _):
        base = (b * NW + wid) * 128
        pltpu.sync_copy(idx_hbm.at[pl.ds(base, 128)], idx_v)
        pltpu.async_copy(emb_hbm.at[idx_v], rows_v, sem).wait()
        pltpu.sync_copy(rows_v, out_hbm.at[pl.ds(base, 128)])
        return _
    lax.fori_loop(0, nb, blk, None)


BA = 512
NBLK = E // BA


def _attn_body(src_h, dst_h, et_h, ai_h, aj_h,
               exp_h, key_h, denp_h,
               den_sh, src_v, dst_v, et_v, key_v,
               ai_r, aj_r, exr_r, exp0_v, exp1_v, s1, s2, s3, s4, s5):
    c = lax.axis_index("c")
    t = lax.axis_index("s")
    lanes = lax.iota(i32, L)

    def zrow(i, _):
        exr_r.at[i][...] = jnp.zeros((L,), f32)
        return _
    lax.fori_loop(0, BA, zrow, None)
    pltpu.sync_copy(exr_r, den_sh.at[pl.ds(t * TRD, BA)])
    pltpu.sync_copy(exr_r.at[pl.ds(0, TRD - BA)],
                    den_sh.at[pl.ds(t * TRD + BA, TRD - BA)])
    plsc.subcore_barrier()

    wid = c * NS + t

    def block(b, _):
        base = (b * NW + wid) * BA
        l1 = pltpu.async_copy(src_h.at[pl.ds(base, BA)], src_v, s3)
        l2 = pltpu.async_copy(dst_h.at[pl.ds(base, BA)], dst_v, s4)
        l3 = pltpu.async_copy(et_h.at[pl.ds(base, BA)], et_v, s5)
        l1.wait()
        l2.wait()
        l3.wait()

        def keyloop(j, _):
            sl = pl.ds(j * L, L)
            key_v[sl] = et_v[sl] * N + dst_v[sl]
            return _
        lax.fori_loop(0, BA // L, keyloop, None)
        cp1 = pltpu.async_copy(ai_h.at[dst_v], ai_r, s1)
        cp2 = pltpu.async_copy(aj_h.at[src_v], aj_r, s2)
        cp1.wait()
        cp2.wait()

        landc = lanes & -2

        def exloop(j, _):
            etv = et_v[pl.ds(j * L, L)]
            acc0 = jnp.zeros((L,), f32)
            acc1 = jnp.zeros((L,), f32)
            for l in range(L):
                e = j * L + l
                etu = _lane_take(etv, jnp.full((L,), l, i32))
                col = etu * H
                v = ai_r.at[e][...] + aj_r.at[e][...]
                exv = jnp.exp(jnp.maximum(v, NEG * v))
                exr_r.at[e][...] = jnp.where(landc == col, exv, 0.0)
                m = lanes == l
                acc0 = jnp.where(m, _lane_take(exv, col), acc0)
                acc1 = jnp.where(m, _lane_take(exv, col + 1), acc1)
            exp0_v[pl.ds(j * L, L)] = acc0
            exp1_v[pl.ds(j * L, L)] = acc1
            return _
        lax.fori_loop(0, BA // L, exloop, None)
        pltpu.sync_copy(exr_r, den_sh.at[dst_v], add=True)
        pltpu.sync_copy(exp0_v, exp_h.at[0].at[pl.ds(base, BA)])
        pltpu.sync_copy(exp1_v, exp_h.at[1].at[pl.ds(base, BA)])
        pltpu.sync_copy(key_v, key_h.at[pl.ds(base, BA)])
        return _
    lax.fori_loop(0, (NBLK - wid + NW - 1) // NW, block, None)

    plsc.subcore_barrier()
    pltpu.sync_copy(den_sh.at[pl.ds(t * TRD, TRD)],
                    denp_h.at[c].at[pl.ds(t * TRD, TRD)])


def _make_attn():
    return pl.kernel(
        _attn_body,
        out_type=(jax.ShapeDtypeStruct((2, E), f32),
                  jax.ShapeDtypeStruct((E,), i32),
                  jax.ShapeDtypeStruct((NC, NP, 16), f32)),
        mesh=_mesh(),
        compiler_params=pltpu.CompilerParams(use_tc_tiling_on_sc=False),
        scratch_types=[
            pltpu.VMEM_SHARED((NP, 16), f32),
            pltpu.VMEM((BA,), i32), pltpu.VMEM((BA,), i32),
            pltpu.VMEM((BA,), i32), pltpu.VMEM((BA,), i32),
            pltpu.VMEM((BA, 16), f32), pltpu.VMEM((BA, 16), f32),
            pltpu.VMEM((BA, 16), f32),
            pltpu.VMEM((BA,), f32), pltpu.VMEM((BA,), f32),
            pltpu.SemaphoreType.DMA, pltpu.SemaphoreType.DMA,
            pltpu.SemaphoreType.DMA, pltpu.SemaphoreType.DMA,
            pltpu.SemaphoreType.DMA,
        ])


BB = 256
HB = BB // 2
NBLKB = E // BB


def _msg_body(key_h, src_h, exp_h, hj_h,
              zp_h,
              z_sh, hj_sh, key_v, src_v, exm_v, hj_r, s1, s2, s3, s4):
    c = lax.axis_index("c")
    t = lax.axis_index("s")
    lanes = lax.iota(i32, L)

    for cc in range(4):
        h = cc // 2

        @pl.when(c == h)
        def _chunk():
            def zrow(i, _):
                hj_r.at[i][pl.ds(0, L)] = jnp.zeros((L,), f32)
                hj_r.at[i][pl.ds(L, L)] = jnp.zeros((L,), f32)
                return _
            lax.fori_loop(0, BB, zrow, None)
            for k in range(12):
                pltpu.sync_copy(hj_r, z_sh.at[pl.ds(t * TRK + k * BB, BB)])
            pltpu.sync_copy(hj_r.at[pl.ds(0, 56)],
                            z_sh.at[pl.ds(t * TRK + 12 * BB, 56)])
            pltpu.sync_copy(hj_h.at[pl.ds(t * TRD, TRD), pl.ds(cc * 32, 32)],
                            hj_sh.at[pl.ds(t * TRD, TRD)])
            plsc.subcore_barrier()

            def block(b, _):
                base = (b * NS + t) * BB
                l1 = pltpu.async_copy(key_h.at[pl.ds(base, BB)], key_v, s3)
                l2 = pltpu.async_copy(src_h.at[pl.ds(base, BB)], src_v, s4)
                cp1 = pltpu.async_copy(exp_h.at[h].at[pl.ds(base, BB)],
                                       exm_v, s1)
                l1.wait()
                l2.wait()
                g0 = pltpu.async_copy(hj_sh.at[src_v.at[pl.ds(0, HB)]],
                                      hj_r.at[pl.ds(0, HB)], s2)
                g1 = pltpu.async_copy(hj_sh.at[src_v.at[pl.ds(HB, HB)]],
                                      hj_r.at[pl.ds(HB, HB)], s3)
                cp1.wait()
                g0.wait()

                def scale(j, _):
                    row = exm_v[pl.ds(j * L, L)]
                    for l in range(L):
                        e = j * L + l
                        sv = _lane_take(row, jnp.full((L,), l, i32))
                        r0 = hj_r.at[e][pl.ds(0, L)]
                        r1 = hj_r.at[e][pl.ds(L, L)]
                        hj_r.at[e][pl.ds(0, L)] = r0 * sv
                        hj_r.at[e][pl.ds(L, L)] = r1 * sv
                    return _
                lax.fori_loop(0, HB // L, scale, None)
                sc0 = pltpu.async_copy(hj_r.at[pl.ds(0, HB)],
                                       z_sh.at[key_v.at[pl.ds(0, HB)]],
                                       s4, add=True)
                g1.wait()
                lax.fori_loop(HB // L, BB // L, scale, None)
                pltpu.sync_copy(hj_r.at[pl.ds(HB, HB)],
                                z_sh.at[key_v.at[pl.ds(HB, HB)]], add=True)
                sc0.wait()
                return _
            lax.fori_loop(0, (NBLKB - t + NS - 1) // NS, block, None)

            plsc.subcore_barrier()
            pltpu.sync_copy(z_sh.at[pl.ds(t * TRK, TRK)],
                            zp_h.at[pl.ds(t * TRK, TRK), pl.ds(cc * 32, 32)])
            plsc.subcore_barrier()


def _make_msg():
    return pl.kernel(
        _msg_body,
        out_type=jax.ShapeDtypeStruct((KS, 128), f32),
        mesh=_mesh(),
        compiler_params=pltpu.CompilerParams(use_tc_tiling_on_sc=False),
        scratch_types=[
            pltpu.VMEM_SHARED((KS, 32), f32),
            pltpu.VMEM_SHARED((NP, 32), f32),
            pltpu.VMEM((BB,), i32), pltpu.VMEM((BB,), i32),
            pltpu.VMEM((BB,), f32), pltpu.VMEM((BB, 32), f32),
            pltpu.SemaphoreType.DMA, pltpu.SemaphoreType.DMA,
            pltpu.SemaphoreType.DMA, pltpu.SemaphoreType.DMA,
        ])


def _pre_tc(x, wcat):
    M, K = x.shape
    bn = 1024
    grid = (M // bn,)

    def body(x_ref, w_ref, hj, sn, ai, aj, xs):
        y = jnp.dot(x_ref[...], w_ref[...], preferred_element_type=f32)
        hj[...] = y[:, 0:128]
        sn[...] = y[:, 128:256]
        ai[...] = y[:, 256:272]
        aj[...] = y[:, 272:288]
        xs[...] = y[:, 288:352]

    outs = [
        jax.ShapeDtypeStruct((M, 128), f32),
        jax.ShapeDtypeStruct((M, 128), f32),
        jax.ShapeDtypeStruct((M, 16), f32),
        jax.ShapeDtypeStruct((M, 16), f32),
        jax.ShapeDtypeStruct((M, 64), f32),
    ]
    ospec = [pl.BlockSpec((bn, s.shape[1]), lambda i: (i, 0)) for s in outs]
    return pl.pallas_call(
        body,
        grid=grid,
        in_specs=[pl.BlockSpec((bn, K), lambda i: (i, 0)),
                  pl.BlockSpec((K, 352), lambda i: (0, 0))],
        out_specs=ospec,
        out_shape=outs,
    )(x, wcat)


def _post_tc(zp, denp, sn, xs, wqkv, wrel, last):
    bn = 1000
    grid = (N // bn,)
    bsel = (lax.broadcasted_iota(i32, (R * R * 64, 128), 0) // 64 ==
            lax.broadcasted_iota(i32, (R * R * 64, 128), 1)).astype(f32)
    bsum = (lax.broadcasted_iota(i32, (R * 64, 128), 0) // 64 ==
            lax.broadcasted_iota(i32, (R * 64, 128), 1)).astype(f32)

    def body(z0, z1, z2, z3, z4, dn_r, sn_r, xs_r, wq_r, wr_r,
             bsel_r, bsum_r, out):
        zrefs = (z0, z1, z2, z3, z4)
        den = dn_r[0] + dn_r[1]
        den = jnp.where(den > 0, den, 1.0)
        dinv = 1.0 / den
        sn_b = sn_r[...]
        xs_b = xs_r[...]
        qs, ks, vs = [], [], []
        for r in range(R):
            zr = zrefs[r][...]
            d0 = jnp.broadcast_to(dinv[:, 2 * r:2 * r + 1], (bn, 64))
            d1 = jnp.broadcast_to(dinv[:, 2 * r + 1:2 * r + 2], (bn, 64))
            zfull = zr * jnp.concatenate([d0, d1], axis=1) + sn_b
            qkv = jnp.dot(zfull, wq_r[r], preferred_element_type=f32)
            qs.append(qkv[:, 0:64])
            ks.append(qkv[:, 64:128])
            vs.append(qkv[:, 128:192])
        m_cat = jnp.concatenate(
            [qs[r] * ks[s] for r in range(R) for s in range(R)], axis=1)
        es = jnp.exp(jnp.dot(m_cat, bsel_r[...],
                             preferred_element_type=f32))
        deltas = []
        for r in range(R):
            tot = (es[:, 5 * r:5 * r + 1] + es[:, 5 * r + 1:5 * r + 2]
                   + es[:, 5 * r + 2:5 * r + 3] + es[:, 5 * r + 3:5 * r + 4]
                   + es[:, 5 * r + 4:5 * r + 5])
            ti = 1.0 / tot
            deltas.append(sum((es[:, 5 * r + s:5 * r + s + 1] * ti) * vs[s]
                              for s in range(R)))
        d_cat = jnp.concatenate(deltas, axis=1)
        dsums = jnp.dot(d_cat, bsum_r[...], preferred_element_type=f32)
        acc = jnp.zeros((bn, 64), f32)
        for r in range(R):
            m = (dsums[:, r:r + 1] != 0).astype(f32)
            acc = acc + (deltas[r] + xs_b * m) * wr_r[r:r + 1, 0:1]
        if last:
            mx = jnp.max(acc, axis=1, keepdims=True)
            lse = mx + jnp.log(jnp.sum(jnp.exp(acc - mx), axis=1,
                                       keepdims=True))
            out[...] = acc - lse
        else:
            out[...] = jnp.maximum(acc, 0.0)

    def zspec(r):
        return pl.BlockSpec((bn, 128),
                            lambda i, r=r: (r * (N // bn) + i, 0))

    return pl.pallas_call(
        body,
        grid=grid,
        in_specs=[zspec(0), zspec(1), zspec(2), zspec(3), zspec(4),
                  pl.BlockSpec((NC, bn, 16), lambda i: (0, i, 0)),
                  pl.BlockSpec((bn, 128), lambda i: (i, 0)),
                  pl.BlockSpec((bn, 64), lambda i: (i, 0)),
                  pl.BlockSpec((R, 128, 192), lambda i: (0, 0, 0)),
                  pl.BlockSpec((8, 128), lambda i: (0, 0)),
                  pl.BlockSpec((R * R * 64, 128), lambda i: (0, 0)),
                  pl.BlockSpec((R * 64, 128), lambda i: (0, 0))],
        out_specs=pl.BlockSpec((bn, 64), lambda i: (i, 0)),
        out_shape=jax.ShapeDtypeStruct((N, 64), f32),
    )(zp, zp, zp, zp, zp, denp, sn, xs, wqkv, wrel, bsel, bsum)


def _fold_weights(p, out_c):
    att = p['node_att']
    att_i = att[:, :, :out_c]
    att_j = att[:, :, out_c:]
    eye = jnp.eye(H, dtype=f32)
    Mi = jnp.einsum('rhc,hk->hcrk', att_i, eye).reshape(H * out_c, R * H)
    Mj = jnp.einsum('rhc,hk->hcrk', att_j, eye).reshape(H * out_c, R * H)
    pad = jnp.zeros((p['lin_i'].shape[0], 16 - R * H), f32)
    w_ai = jnp.concatenate([p['lin_i'] @ Mi, pad], axis=1)
    w_aj = jnp.concatenate([p['lin_j'] @ Mj, pad], axis=1)
    wcat = jnp.concatenate(
        [p['lin_j'], p['W_self_node'], w_ai, w_aj, p['W_self']], axis=1)
    wqkv = jnp.concatenate([p['W_q'], p['W_k'], p['W_v']], axis=2)
    wrel = jnp.zeros((8, 128), f32).at[:R, 0].set(p['W_relation'][:, 0])
    return wcat, wqkv, wrel


_attn_call = _make_attn()
_msg_call = _make_msg()


def kernel(n_id, local_node_idx, edge_index, edge_type, node_type, emb,
           params):
    src = edge_index[0]
    dst = edge_index[1]
    wcat1, wqkv1, wrel1 = _fold_weights(params[0], 64)
    wcat2, wqkv2, wrel2 = _fold_weights(params[1], 64)

    idxp = jnp.concatenate([local_node_idx, jnp.zeros((NP - N,), i32)])
    x = _gather_rows(emb, idxp)

    for li, (wcat, wqkv, wrel) in enumerate(
            [(wcat1, wqkv1, wrel1), (wcat2, wqkv2, wrel2)]):
        hj, sn, ai, aj, xs = _pre_tc(x, wcat)
        exr, key, denp = _attn_call(src, dst, edge_type, ai, aj)
        zp = _msg_call(key, src, exr, hj)
        x = _post_tc(zp, denp, sn, xs, wqkv, wrel, last=(li == 1))
        if li == 0:
            x = jnp.concatenate([x, jnp.zeros((NP - N, 64), f32)], axis=0)
    return x
